# Initial kernel scaffold; baseline (speedup 1.0000x reference)
import jax
import jax.numpy as jnp
from jax.experimental import pallas as pl


def kernel(points, normals, distances, w1a, b1a, w1b, b1b, w2a, b2a, w2b, b2b, w3a, b3a, w3b, b3b, w4, b4, w5, b5):
    raise NotImplementedError("write your pallas kernel here")


# speedup vs baseline: 5.6271x; 5.6271x over previous
import functools

import jax
import jax.numpy as jnp
from jax import lax
from jax.experimental import pallas as pl
from jax.experimental.pallas import tpu as pltpu
from jax.experimental.pallas import tpu_sc as plsc

NPTS = 10000
KNN = 16
NPLANES = 8
PLANE_THR = 0.05


KNN_R = 80


def _knn_body(pts_ref, ptsT_ref, wA_ref, wB_ref, ba_ref,
              nbr_ref, ax_ref, bx_ref):
    i = pl.program_id(0)
    pts_r = pts_ref[...]
    ptsT = ptsT_ref[...]

    sq_r = jnp.zeros((KNN_R, 1), jnp.float32)
    sq_all = jnp.zeros((1, NPTS), jnp.float32)
    for c in range(3):
        sq_r = sq_r + pts_r[:, c:c + 1] * pts_r[:, c:c + 1]
        sq_all = sq_all + ptsT[c:c + 1, :] * ptsT[c:c + 1, :]
    dot = jnp.dot(pts_r.astype(jnp.bfloat16), ptsT.astype(jnp.bfloat16),
                  preferred_element_type=jnp.float32)
    d = (sq_r + sq_all) - 2.0 * dot

    col = lax.broadcasted_iota(jnp.int32, (KNN_R, NPTS), 1)
    row_global = lax.broadcasted_iota(jnp.int32, (KNN_R, NPTS), 0) + i * KNN_R
    inf = jnp.float32(jnp.inf)
    d = jnp.where(col == row_global, inf, d)

    big = jnp.int32(2**30)
    cols_out = []
    for _ in range(KNN):
        minv = jnp.min(d, axis=1, keepdims=True)
        eq = d == minv
        idx = jnp.min(jnp.where(eq, col, big), axis=1)
        cols_out.append(idx[:, None])
        d = jnp.where(eq, inf, d)
    nbr_ref[...] = jnp.concatenate(cols_out, axis=1)

    ax = jnp.zeros((KNN_R, 64), jnp.float32) + ba_ref[...]
    bx = jnp.zeros((KNN_R, 64), jnp.float32)
    for c in range(3):
        ax = ax + pts_r[:, c:c + 1] * wA_ref[c:c + 1, :]
        bx = bx + pts_r[:, c:c + 1] * wB_ref[c:c + 1, :]
    ax_ref[...] = ax
    bx_ref[...] = bx


def _knn_call(points, pointsT, wA1, wB1, b1a):
    nblk = NPTS // KNN_R
    return pl.pallas_call(
        _knn_body,
        grid=(nblk,),
        in_specs=[
            pl.BlockSpec((KNN_R, 3), lambda i: (i, 0)),
            pl.BlockSpec((3, NPTS), lambda i: (0, 0)),
            pl.BlockSpec((3, 64), lambda i: (0, 0)),
            pl.BlockSpec((3, 64), lambda i: (0, 0)),
            pl.BlockSpec((1, 64), lambda i: (0, 0)),
        ],
        out_specs=[
            pl.BlockSpec((KNN_R, KNN), lambda i: (i, 0)),
            pl.BlockSpec((KNN_R, 64), lambda i: (i, 0)),
            pl.BlockSpec((KNN_R, 64), lambda i: (i, 0)),
        ],
        out_shape=[
            jax.ShapeDtypeStruct((NPTS, KNN), jnp.int32),
            jax.ShapeDtypeStruct((NPTS, 64), jnp.float32),
            jax.ShapeDtypeStruct((NPTS, 64), jnp.float32),
        ],
    )(points, pointsT, wA1, wB1, b1a)


IDX_ROWS = (NPTS * KNN) // 128


def _sc_gather(table, idx2d):
    info = plsc.get_sparse_core_info()
    nc, ns = info.num_cores, info.num_subcores
    nw = nc * ns
    jmax = (IDX_ROWS + nw - 1) // nw
    mesh = plsc.VectorSubcoreMesh(core_axis_name="c", subcore_axis_name="s")

    @functools.partial(
        pl.kernel, mesh=mesh,
        compiler_params=pltpu.CompilerParams(use_tc_tiling_on_sc=False),
        out_type=jax.ShapeDtypeStruct((IDX_ROWS, 128, 64), jnp.float32),
        scratch_types=[
            pltpu.VMEM((128,), jnp.int32),
            pltpu.VMEM((128, 64), jnp.float32),
            pltpu.SemaphoreType.DMA,
        ],
    )
    def gk(table_hbm, idx_hbm, out_hbm, idx_v, rows_v, sem):
        w = lax.axis_index("s") * nc + lax.axis_index("c")

        def body(j, carry):
            row = w + j * nw

            @pl.when(row < IDX_ROWS)
            def _():
                pltpu.sync_copy(idx_hbm.at[row], idx_v)
                pltpu.async_copy(table_hbm.at[idx_v], rows_v, sem).wait()
                pltpu.sync_copy(rows_v, out_hbm.at[row])
            return carry

        lax.fori_loop(0, jmax, body, 0)

    return gk(table, idx2d)


CONV_R = 400


def _convB_A_body(ax_ref, g_ref, wb_ref, bb_ref, wAn_ref, wBn_ref, ban_ref,
                  f_ref, axn_ref, bxn_ref):
    ax = ax_ref[...]
    g = g_ref[...]
    h1 = jax.nn.relu(ax[:, None, :] + g)
    h1f = h1.reshape(CONV_R * KNN, 64)
    h2 = jnp.dot(h1f, wb_ref[...],
                 preferred_element_type=jnp.float32) + bb_ref[...]
    h3 = h2.reshape(CONV_R, KNN, 64)
    f = h3[:, 0, :]
    for k in range(1, KNN):
        f = jnp.maximum(f, h3[:, k, :])
    f_ref[...] = f
    axn_ref[...] = jnp.dot(f, wAn_ref[...],
                           preferred_element_type=jnp.float32) + ban_ref[...]
    bxn_ref[...] = jnp.dot(f, wBn_ref[...], preferred_element_type=jnp.float32)


def _convB_A_call(ax, g3, wb, bb, wAn, wBn, ban):
    nblk = NPTS // CONV_R
    return pl.pallas_call(
        _convB_A_body,
        grid=(nblk,),
        in_specs=[
            pl.BlockSpec((CONV_R, 64), lambda i: (i, 0)),
            pl.BlockSpec((CONV_R, KNN, 64), lambda i: (i, 0, 0)),
            pl.BlockSpec((64, 64), lambda i: (0, 0)),
            pl.BlockSpec((1, 64), lambda i: (0, 0)),
            pl.BlockSpec((64, 64), lambda i: (0, 0)),
            pl.BlockSpec((64, 64), lambda i: (0, 0)),
            pl.BlockSpec((1, 64), lambda i: (0, 0)),
        ],
        out_specs=[
            pl.BlockSpec((CONV_R, 64), lambda i: (i, 0)),
            pl.BlockSpec((CONV_R, 64), lambda i: (i, 0)),
            pl.BlockSpec((CONV_R, 64), lambda i: (i, 0)),
        ],
        out_shape=[
            jax.ShapeDtypeStruct((NPTS, 64), jnp.float32),
            jax.ShapeDtypeStruct((NPTS, 64), jnp.float32),
            jax.ShapeDtypeStruct((NPTS, 64), jnp.float32),
        ],
    )(ax, g3, wb, bb, wAn, wBn, ban)


def _final_body(ax_ref, g_ref, wb_ref, bb_ref, f1_ref, f2_ref,
                w4a_ref, w4b_ref, w4c_ref, b4_ref, w5_ref, b5_ref,
                pts_ref, nT_ref, dist_ref,
                out_ref, cnt_ref, s1_ref):
    i = pl.program_id(0)
    ax = ax_ref[...]
    g = g_ref[...]
    h1 = jax.nn.relu(ax[:, None, :] + g)
    h1f = h1.reshape(CONV_R * KNN, 64)
    h2 = jnp.dot(h1f, wb_ref[...],
                 preferred_element_type=jnp.float32) + bb_ref[...]
    h3 = h2.reshape(CONV_R, KNN, 64)
    f3 = h3[:, 0, :]
    for k in range(1, KNN):
        f3 = jnp.maximum(f3, h3[:, k, :])

    t = (jnp.dot(f1_ref[...], w4a_ref[...], preferred_element_type=jnp.float32)
         + jnp.dot(f2_ref[...], w4b_ref[...], preferred_element_type=jnp.float32)
         + jnp.dot(f3, w4c_ref[...], preferred_element_type=jnp.float32)
         + b4_ref[...])
    t = jax.nn.relu(t)
    res = jnp.dot(t, w5_ref[...], preferred_element_type=jnp.float32) + b5_ref[...]
    pts = pts_ref[...] + res
    out_ref[...] = pts

    nT = nT_ref[...]
    pd = jnp.zeros((CONV_R, NPLANES), jnp.float32) - dist_ref[...]
    for c in range(3):
        pd = pd + pts[:, c:c + 1] * nT[c:c + 1, :]
    m = (jnp.abs(pd) < PLANE_THR).astype(jnp.float32)

    @pl.when(i == 0)
    def _():
        cnt_ref[...] = jnp.zeros_like(cnt_ref)
        s1_ref[...] = jnp.zeros_like(s1_ref)

    cnt_ref[0, :] += jnp.sum(m, axis=0)
    for c in range(3):
        s1_ref[c, :] += jnp.sum(m * pts[:, c:c + 1], axis=0)


def _final_call(ax3, g3, w3b, b3b, f1, f2, w4a, w4b, w4c, b4, w5, b5,
                points, nT, dist):
    nblk = NPTS // CONV_R
    return pl.pallas_call(
        _final_body,
        grid=(nblk,),
        in_specs=[
            pl.BlockSpec((CONV_R, 64), lambda i: (i, 0)),
            pl.BlockSpec((CONV_R, KNN, 64), lambda i: (i, 0, 0)),
            pl.BlockSpec((64, 64), lambda i: (0, 0)),
            pl.BlockSpec((1, 64), lambda i: (0, 0)),
            pl.BlockSpec((CONV_R, 64), lambda i: (i, 0)),
            pl.BlockSpec((CONV_R, 64), lambda i: (i, 0)),
            pl.BlockSpec((64, 256), lambda i: (0, 0)),
            pl.BlockSpec((64, 256), lambda i: (0, 0)),
            pl.BlockSpec((64, 256), lambda i: (0, 0)),
            pl.BlockSpec((1, 256), lambda i: (0, 0)),
            pl.BlockSpec((256, 3), lambda i: (0, 0)),
            pl.BlockSpec((1, 3), lambda i: (0, 0)),
            pl.BlockSpec((CONV_R, 3), lambda i: (i, 0)),
            pl.BlockSpec((3, NPLANES), lambda i: (0, 0)),
            pl.BlockSpec((1, NPLANES), lambda i: (0, 0)),
        ],
        out_specs=[
            pl.BlockSpec((CONV_R, 3), lambda i: (i, 0)),
            pl.BlockSpec((1, NPLANES), lambda i: (0, 0)),
            pl.BlockSpec((3, NPLANES), lambda i: (0, 0)),
        ],
        out_shape=[
            jax.ShapeDtypeStruct((NPTS, 3), jnp.float32),
            jax.ShapeDtypeStruct((1, NPLANES), jnp.float32),
            jax.ShapeDtypeStruct((3, NPLANES), jnp.float32),
        ],
    )(ax3, g3, w3b, b3b, f1, f2, w4a, w4b, w4c, b4, w5, b5, points, nT, dist)


def _cov_body(pts_ref, nT_ref, dist_ref, cT_ref, cov_ref):
    i = pl.program_id(0)
    pts = pts_ref[...]
    nT = nT_ref[...]
    cT = cT_ref[...]
    pd = jnp.zeros((CONV_R, NPLANES), jnp.float32) - dist_ref[...]
    for c in range(3):
        pd = pd + pts[:, c:c + 1] * nT[c:c + 1, :]
    m = (jnp.abs(pd) < PLANE_THR).astype(jnp.float32)

    cen = []
    for c in range(3):
        cen.append((pts[:, c:c + 1] - cT[c:c + 1, :]) * m)

    @pl.when(i == 0)
    def _():
        cov_ref[...] = jnp.zeros_like(cov_ref)

    j = 0
    for a in range(3):
        for b in range(3):
            cov_ref[j, :] += jnp.sum(cen[a] * cen[b], axis=0)
            j += 1


def _cov_call(pts, nT, dist, cT):
    nblk = NPTS // CONV_R
    return pl.pallas_call(
        _cov_body,
        grid=(nblk,),
        in_specs=[
            pl.BlockSpec((CONV_R, 3), lambda i: (i, 0)),
            pl.BlockSpec((3, NPLANES), lambda i: (0, 0)),
            pl.BlockSpec((1, NPLANES), lambda i: (0, 0)),
            pl.BlockSpec((3, NPLANES), lambda i: (0, 0)),
        ],
        out_specs=pl.BlockSpec((9, NPLANES), lambda i: (0, 0)),
        out_shape=jax.ShapeDtypeStruct((9, NPLANES), jnp.float32),
    )(pts, nT, dist, cT)


def _proj_body(pts_ref, nT_ref, dist_ref, rnT_ref, rd_ref, valid_ref, out_ref):
    pts = pts_ref[...]
    nT = nT_ref[...]
    rnT = rnT_ref[...]
    rd = rd_ref[...]
    valid = valid_ref[...]
    pd = jnp.zeros((CONV_R, NPLANES), jnp.float32) - dist_ref[...]
    for c in range(3):
        pd = pd + pts[:, c:c + 1] * nT[c:c + 1, :]
    m = (jnp.abs(pd) < PLANE_THR).astype(jnp.float32)

    px = pts[:, 0]
    py = pts[:, 1]
    pz = pts[:, 2]
    for p in range(NPLANES):
        coef = valid[0, p] * m[:, p]
        dot = px * rnT[0, p] + py * rnT[1, p] + pz * rnT[2, p]
        scale = coef * (dot - rd[0, p])
        px = px - scale * rnT[0, p]
        py = py - scale * rnT[1, p]
        pz = pz - scale * rnT[2, p]
    out_ref[...] = jnp.concatenate(
        [px[:, None], py[:, None], pz[:, None]], axis=1)


def _proj_call(pts, nT, dist, rnT, rd, valid):
    nblk = NPTS // CONV_R
    return pl.pallas_call(
        _proj_body,
        grid=(nblk,),
        in_specs=[
            pl.BlockSpec((CONV_R, 3), lambda i: (i, 0)),
            pl.BlockSpec((3, NPLANES), lambda i: (0, 0)),
            pl.BlockSpec((1, NPLANES), lambda i: (0, 0)),
            pl.BlockSpec((3, NPLANES), lambda i: (0, 0)),
            pl.BlockSpec((1, NPLANES), lambda i: (0, 0)),
            pl.BlockSpec((1, NPLANES), lambda i: (0, 0)),
        ],
        out_specs=pl.BlockSpec((CONV_R, 3), lambda i: (i, 0)),
        out_shape=jax.ShapeDtypeStruct((NPTS, 3), jnp.float32),
    )(pts, nT, dist, rnT, rd, valid)


def kernel(points, normals, distances, w1a, b1a, w1b, b1b, w2a, b2a, w2b, b2b,
           w3a, b3a, w3b, b3b, w4, b4, w5, b5):
    f32 = jnp.float32
    pointsT = points.T
    wA1, wB1 = w1a[:3] - w1a[3:], w1a[3:]
    wA2, wB2 = w2a[:64] - w2a[64:], w2a[64:]
    wA3, wB3 = w3a[:64] - w3a[64:], w3a[64:]

    nbrs, ax1, bx1 = _knn_call(points, pointsT, wA1, wB1, b1a[None, :])
    idx2d = nbrs.reshape(IDX_ROWS, 128)

    g1 = _sc_gather(bx1, idx2d).reshape(NPTS, KNN, 64)
    f1, ax2, bx2 = _convB_A_call(ax1, g1, w1b, b1b[None, :],
                                 wA2, wB2, b2a[None, :])
    g2 = _sc_gather(bx2, idx2d).reshape(NPTS, KNN, 64)
    f2, ax3, bx3 = _convB_A_call(ax2, g2, w2b, b2b[None, :],
                                 wA3, wB3, b3a[None, :])
    g3 = _sc_gather(bx3, idx2d).reshape(NPTS, KNN, 64)

    nT = normals.T.astype(f32)
    dist = distances[None, :].astype(f32)
    pts, cnt2d, s1 = _final_call(
        ax3, g3, w3b, b3b[None, :], f1, f2,
        w4[0:64], w4[64:128], w4[128:192], b4[None, :], w5, b5[None, :],
        points, nT, dist)

    cnt = cnt2d[0]
    centroid = (s1 / jnp.maximum(cnt, 1.0)[None, :])
    cov = _cov_call(pts, nT, dist, centroid)
    covm = cov.T.reshape(NPLANES, 3, 3)
    _, _, vh = jnp.linalg.svd(covm, full_matrices=False)
    rn = vh[:, 2, :]
    flip = jnp.sum(rn * normals, axis=1) < 0.0
    rn = jnp.where(flip[:, None], -rn, rn)
    rd = jnp.sum(centroid.T * rn, axis=1)
    valid = (cnt >= 3.0).astype(f32)

    return _proj_call(pts, nT, dist, rn.T, rd[None, :], valid[None, :])


# speedup vs baseline: 5.9867x; 1.0639x over previous
import functools

import jax
import jax.numpy as jnp
from jax import lax
from jax.experimental import pallas as pl
from jax.experimental.pallas import tpu as pltpu
from jax.experimental.pallas import tpu_sc as plsc

NPTS = 10000
KNN = 16
NPLANES = 8
PLANE_THR = 0.05


KNN_R = 80


def _knn_body(pts_ref, ptsT_ref, wA_ref, wB_ref, ba_ref,
              nbr_ref, ax_ref, bx_ref):
    i = pl.program_id(0)
    pts_r = pts_ref[...]
    ptsT = ptsT_ref[...]

    sq_r = jnp.zeros((KNN_R, 1), jnp.float32)
    sq_all = jnp.zeros((1, NPTS), jnp.float32)
    for c in range(3):
        sq_r = sq_r + pts_r[:, c:c + 1] * pts_r[:, c:c + 1]
        sq_all = sq_all + ptsT[c:c + 1, :] * ptsT[c:c + 1, :]
    dot = jnp.dot(pts_r.astype(jnp.bfloat16), ptsT.astype(jnp.bfloat16),
                  preferred_element_type=jnp.float32)
    d = (sq_r + sq_all) - 2.0 * dot

    colf = lax.broadcasted_iota(jnp.int32, (KNN_R, NPTS), 1).astype(jnp.float32)
    row_global = (lax.broadcasted_iota(jnp.int32, (KNN_R, NPTS), 0)
                  .astype(jnp.float32) + jnp.float32(i * KNN_R))
    inf = jnp.float32(jnp.inf)
    d = jnp.where(colf == row_global, inf, d)

    bigf = jnp.float32(2**30)
    cols_out = []
    minv = jnp.min(d, axis=1, keepdims=True)
    for k in range(KNN):
        eq = d == minv
        idxf = jnp.min(jnp.where(eq, colf, bigf), axis=1)
        cols_out.append(idxf[:, None])
        if k < KNN - 1:
            d = jnp.where(eq, inf, d)
            minv = jnp.min(d, axis=1, keepdims=True)
    nbr_ref[...] = jnp.concatenate(cols_out, axis=1).astype(jnp.int32)

    ax = jnp.zeros((KNN_R, 64), jnp.float32) + ba_ref[...]
    bx = jnp.zeros((KNN_R, 64), jnp.float32)
    for c in range(3):
        ax = ax + pts_r[:, c:c + 1] * wA_ref[c:c + 1, :]
        bx = bx + pts_r[:, c:c + 1] * wB_ref[c:c + 1, :]
    ax_ref[...] = ax
    bx_ref[...] = bx


def _knn_call(points, pointsT, wA1, wB1, b1a):
    nblk = NPTS // KNN_R
    return pl.pallas_call(
        _knn_body,
        grid=(nblk,),
        in_specs=[
            pl.BlockSpec((KNN_R, 3), lambda i: (i, 0)),
            pl.BlockSpec((3, NPTS), lambda i: (0, 0)),
            pl.BlockSpec((3, 64), lambda i: (0, 0)),
            pl.BlockSpec((3, 64), lambda i: (0, 0)),
            pl.BlockSpec((1, 64), lambda i: (0, 0)),
        ],
        out_specs=[
            pl.BlockSpec((KNN_R, KNN), lambda i: (i, 0)),
            pl.BlockSpec((KNN_R, 64), lambda i: (i, 0)),
            pl.BlockSpec((KNN_R, 64), lambda i: (i, 0)),
        ],
        out_shape=[
            jax.ShapeDtypeStruct((NPTS, KNN), jnp.int32),
            jax.ShapeDtypeStruct((NPTS, 64), jnp.float32),
            jax.ShapeDtypeStruct((NPTS, 64), jnp.float32),
        ],
    )(points, pointsT, wA1, wB1, b1a)


IDX_ROWS = (NPTS * KNN) // 128


def _sc_gather(table, idx2d):
    info = plsc.get_sparse_core_info()
    nc, ns = info.num_cores, info.num_subcores
    nw = nc * ns
    jmax = (IDX_ROWS + nw - 1) // nw
    mesh = plsc.VectorSubcoreMesh(core_axis_name="c", subcore_axis_name="s")

    @functools.partial(
        pl.kernel, mesh=mesh,
        compiler_params=pltpu.CompilerParams(use_tc_tiling_on_sc=False),
        out_type=jax.ShapeDtypeStruct((IDX_ROWS, 128, 64), jnp.float32),
        scratch_types=[
            pltpu.VMEM((128,), jnp.int32),
            pltpu.VMEM((128, 64), jnp.float32),
            pltpu.SemaphoreType.DMA,
        ],
    )
    def gk(table_hbm, idx_hbm, out_hbm, idx_v, rows_v, sem):
        w = lax.axis_index("s") * nc + lax.axis_index("c")

        def body(j, carry):
            row = w + j * nw

            @pl.when(row < IDX_ROWS)
            def _():
                pltpu.sync_copy(idx_hbm.at[row], idx_v)
                pltpu.async_copy(table_hbm.at[idx_v], rows_v, sem).wait()
                pltpu.sync_copy(rows_v, out_hbm.at[row])
            return carry

        lax.fori_loop(0, jmax, body, 0)

    return gk(table, idx2d)


CONV_R = 400


def _convB_A_body(ax_ref, g_ref, wb_ref, bb_ref, wAn_ref, wBn_ref, ban_ref,
                  f_ref, axn_ref, bxn_ref):
    ax = ax_ref[...]
    g = g_ref[...]
    h1 = jax.nn.relu(ax[:, None, :] + g)
    h1f = h1.reshape(CONV_R * KNN, 64)
    h2 = jnp.dot(h1f, wb_ref[...],
                 preferred_element_type=jnp.float32) + bb_ref[...]
    h3 = h2.reshape(CONV_R, KNN, 64)
    f = h3[:, 0, :]
    for k in range(1, KNN):
        f = jnp.maximum(f, h3[:, k, :])
    f_ref[...] = f
    axn_ref[...] = jnp.dot(f, wAn_ref[...],
                           preferred_element_type=jnp.float32) + ban_ref[...]
    bxn_ref[...] = jnp.dot(f, wBn_ref[...], preferred_element_type=jnp.float32)


def _convB_A_call(ax, g3, wb, bb, wAn, wBn, ban):
    nblk = NPTS // CONV_R
    return pl.pallas_call(
        _convB_A_body,
        grid=(nblk,),
        in_specs=[
            pl.BlockSpec((CONV_R, 64), lambda i: (i, 0)),
            pl.BlockSpec((CONV_R, KNN, 64), lambda i: (i, 0, 0)),
            pl.BlockSpec((64, 64), lambda i: (0, 0)),
            pl.BlockSpec((1, 64), lambda i: (0, 0)),
            pl.BlockSpec((64, 64), lambda i: (0, 0)),
            pl.BlockSpec((64, 64), lambda i: (0, 0)),
            pl.BlockSpec((1, 64), lambda i: (0, 0)),
        ],
        out_specs=[
            pl.BlockSpec((CONV_R, 64), lambda i: (i, 0)),
            pl.BlockSpec((CONV_R, 64), lambda i: (i, 0)),
            pl.BlockSpec((CONV_R, 64), lambda i: (i, 0)),
        ],
        out_shape=[
            jax.ShapeDtypeStruct((NPTS, 64), jnp.float32),
            jax.ShapeDtypeStruct((NPTS, 64), jnp.float32),
            jax.ShapeDtypeStruct((NPTS, 64), jnp.float32),
        ],
    )(ax, g3, wb, bb, wAn, wBn, ban)


def _final_body(ax_ref, g_ref, wb_ref, bb_ref, f1_ref, f2_ref,
                w4a_ref, w4b_ref, w4c_ref, b4_ref, w5_ref, b5_ref,
                pts_ref, nT_ref, dist_ref,
                out_ref, cnt_ref, s1_ref):
    i = pl.program_id(0)
    ax = ax_ref[...]
    g = g_ref[...]
    h1 = jax.nn.relu(ax[:, None, :] + g)
    h1f = h1.reshape(CONV_R * KNN, 64)
    h2 = jnp.dot(h1f, wb_ref[...],
                 preferred_element_type=jnp.float32) + bb_ref[...]
    h3 = h2.reshape(CONV_R, KNN, 64)
    f3 = h3[:, 0, :]
    for k in range(1, KNN):
        f3 = jnp.maximum(f3, h3[:, k, :])

    t = (jnp.dot(f1_ref[...], w4a_ref[...], preferred_element_type=jnp.float32)
         + jnp.dot(f2_ref[...], w4b_ref[...], preferred_element_type=jnp.float32)
         + jnp.dot(f3, w4c_ref[...], preferred_element_type=jnp.float32)
         + b4_ref[...])
    t = jax.nn.relu(t)
    res = jnp.dot(t, w5_ref[...], preferred_element_type=jnp.float32) + b5_ref[...]
    pts = pts_ref[...] + res
    out_ref[...] = pts

    nT = nT_ref[...]
    pd = jnp.zeros((CONV_R, NPLANES), jnp.float32) - dist_ref[...]
    for c in range(3):
        pd = pd + pts[:, c:c + 1] * nT[c:c + 1, :]
    m = (jnp.abs(pd) < PLANE_THR).astype(jnp.float32)

    @pl.when(i == 0)
    def _():
        cnt_ref[...] = jnp.zeros_like(cnt_ref)
        s1_ref[...] = jnp.zeros_like(s1_ref)

    cnt_ref[0, :] += jnp.sum(m, axis=0)
    for c in range(3):
        s1_ref[c, :] += jnp.sum(m * pts[:, c:c + 1], axis=0)


def _final_call(ax3, g3, w3b, b3b, f1, f2, w4a, w4b, w4c, b4, w5, b5,
                points, nT, dist):
    nblk = NPTS // CONV_R
    return pl.pallas_call(
        _final_body,
        grid=(nblk,),
        in_specs=[
            pl.BlockSpec((CONV_R, 64), lambda i: (i, 0)),
            pl.BlockSpec((CONV_R, KNN, 64), lambda i: (i, 0, 0)),
            pl.BlockSpec((64, 64), lambda i: (0, 0)),
            pl.BlockSpec((1, 64), lambda i: (0, 0)),
            pl.BlockSpec((CONV_R, 64), lambda i: (i, 0)),
            pl.BlockSpec((CONV_R, 64), lambda i: (i, 0)),
            pl.BlockSpec((64, 256), lambda i: (0, 0)),
            pl.BlockSpec((64, 256), lambda i: (0, 0)),
            pl.BlockSpec((64, 256), lambda i: (0, 0)),
            pl.BlockSpec((1, 256), lambda i: (0, 0)),
            pl.BlockSpec((256, 3), lambda i: (0, 0)),
            pl.BlockSpec((1, 3), lambda i: (0, 0)),
            pl.BlockSpec((CONV_R, 3), lambda i: (i, 0)),
            pl.BlockSpec((3, NPLANES), lambda i: (0, 0)),
            pl.BlockSpec((1, NPLANES), lambda i: (0, 0)),
        ],
        out_specs=[
            pl.BlockSpec((CONV_R, 3), lambda i: (i, 0)),
            pl.BlockSpec((1, NPLANES), lambda i: (0, 0)),
            pl.BlockSpec((3, NPLANES), lambda i: (0, 0)),
        ],
        out_shape=[
            jax.ShapeDtypeStruct((NPTS, 3), jnp.float32),
            jax.ShapeDtypeStruct((1, NPLANES), jnp.float32),
            jax.ShapeDtypeStruct((3, NPLANES), jnp.float32),
        ],
    )(ax3, g3, w3b, b3b, f1, f2, w4a, w4b, w4c, b4, w5, b5, points, nT, dist)


def _cov_body(pts_ref, nT_ref, dist_ref, cT_ref, cov_ref):
    i = pl.program_id(0)
    pts = pts_ref[...]
    nT = nT_ref[...]
    cT = cT_ref[...]
    pd = jnp.zeros((CONV_R, NPLANES), jnp.float32) - dist_ref[...]
    for c in range(3):
        pd = pd + pts[:, c:c + 1] * nT[c:c + 1, :]
    m = (jnp.abs(pd) < PLANE_THR).astype(jnp.float32)

    cen = []
    for c in range(3):
        cen.append((pts[:, c:c + 1] - cT[c:c + 1, :]) * m)

    @pl.when(i == 0)
    def _():
        cov_ref[...] = jnp.zeros_like(cov_ref)

    j = 0
    for a in range(3):
        for b in range(3):
            cov_ref[j, :] += jnp.sum(cen[a] * cen[b], axis=0)
            j += 1


def _cov_call(pts, nT, dist, cT):
    nblk = NPTS // CONV_R
    return pl.pallas_call(
        _cov_body,
        grid=(nblk,),
        in_specs=[
            pl.BlockSpec((CONV_R, 3), lambda i: (i, 0)),
            pl.BlockSpec((3, NPLANES), lambda i: (0, 0)),
            pl.BlockSpec((1, NPLANES), lambda i: (0, 0)),
            pl.BlockSpec((3, NPLANES), lambda i: (0, 0)),
        ],
        out_specs=pl.BlockSpec((9, NPLANES), lambda i: (0, 0)),
        out_shape=jax.ShapeDtypeStruct((9, NPLANES), jnp.float32),
    )(pts, nT, dist, cT)


def _proj_body(pts_ref, nT_ref, dist_ref, rnT_ref, rd_ref, valid_ref, out_ref):
    pts = pts_ref[...]
    nT = nT_ref[...]
    rnT = rnT_ref[...]
    rd = rd_ref[...]
    valid = valid_ref[...]
    pd = jnp.zeros((CONV_R, NPLANES), jnp.float32) - dist_ref[...]
    for c in range(3):
        pd = pd + pts[:, c:c + 1] * nT[c:c + 1, :]
    m = (jnp.abs(pd) < PLANE_THR).astype(jnp.float32)

    px = pts[:, 0]
    py = pts[:, 1]
    pz = pts[:, 2]
    for p in range(NPLANES):
        coef = valid[0, p] * m[:, p]
        dot = px * rnT[0, p] + py * rnT[1, p] + pz * rnT[2, p]
        scale = coef * (dot - rd[0, p])
        px = px - scale * rnT[0, p]
        py = py - scale * rnT[1, p]
        pz = pz - scale * rnT[2, p]
    out_ref[...] = jnp.concatenate(
        [px[:, None], py[:, None], pz[:, None]], axis=1)


def _proj_call(pts, nT, dist, rnT, rd, valid):
    nblk = NPTS // CONV_R
    return pl.pallas_call(
        _proj_body,
        grid=(nblk,),
        in_specs=[
            pl.BlockSpec((CONV_R, 3), lambda i: (i, 0)),
            pl.BlockSpec((3, NPLANES), lambda i: (0, 0)),
            pl.BlockSpec((1, NPLANES), lambda i: (0, 0)),
            pl.BlockSpec((3, NPLANES), lambda i: (0, 0)),
            pl.BlockSpec((1, NPLANES), lambda i: (0, 0)),
            pl.BlockSpec((1, NPLANES), lambda i: (0, 0)),
        ],
        out_specs=pl.BlockSpec((CONV_R, 3), lambda i: (i, 0)),
        out_shape=jax.ShapeDtypeStruct((NPTS, 3), jnp.float32),
    )(pts, nT, dist, rnT, rd, valid)


def kernel(points, normals, distances, w1a, b1a, w1b, b1b, w2a, b2a, w2b, b2b,
           w3a, b3a, w3b, b3b, w4, b4, w5, b5):
    f32 = jnp.float32
    pointsT = points.T
    wA1, wB1 = w1a[:3] - w1a[3:], w1a[3:]
    wA2, wB2 = w2a[:64] - w2a[64:], w2a[64:]
    wA3, wB3 = w3a[:64] - w3a[64:], w3a[64:]

    nbrs, ax1, bx1 = _knn_call(points, pointsT, wA1, wB1, b1a[None, :])
    idx2d = nbrs.reshape(IDX_ROWS, 128)

    g1 = _sc_gather(bx1, idx2d).reshape(NPTS, KNN, 64)
    f1, ax2, bx2 = _convB_A_call(ax1, g1, w1b, b1b[None, :],
                                 wA2, wB2, b2a[None, :])
    g2 = _sc_gather(bx2, idx2d).reshape(NPTS, KNN, 64)
    f2, ax3, bx3 = _convB_A_call(ax2, g2, w2b, b2b[None, :],
                                 wA3, wB3, b3a[None, :])
    g3 = _sc_gather(bx3, idx2d).reshape(NPTS, KNN, 64)

    nT = normals.T.astype(f32)
    dist = distances[None, :].astype(f32)
    pts, cnt2d, s1 = _final_call(
        ax3, g3, w3b, b3b[None, :], f1, f2,
        w4[0:64], w4[64:128], w4[128:192], b4[None, :], w5, b5[None, :],
        points, nT, dist)

    cnt = cnt2d[0]
    centroid = (s1 / jnp.maximum(cnt, 1.0)[None, :])
    cov = _cov_call(pts, nT, dist, centroid)
    covm = cov.T.reshape(NPLANES, 3, 3)
    _, _, vh = jnp.linalg.svd(covm, full_matrices=False)
    rn = vh[:, 2, :]
    flip = jnp.sum(rn * normals, axis=1) < 0.0
    rn = jnp.where(flip[:, None], -rn, rn)
    rd = jnp.sum(centroid.T * rn, axis=1)
    valid = (cnt >= 3.0).astype(f32)

    return _proj_call(pts, nT, dist, rn.T, rd[None, :], valid[None, :])


# speedup vs baseline: 6.1655x; 1.0299x over previous
import functools

import jax
import jax.numpy as jnp
from jax import lax
from jax.experimental import pallas as pl
from jax.experimental.pallas import tpu as pltpu
from jax.experimental.pallas import tpu_sc as plsc

NPTS = 10000
KNN = 16
NPLANES = 8
PLANE_THR = 0.05


KNN_R = 80


def _knn_body(pts_ref, ptsT_ref, wA_ref, wB_ref, ba_ref,
              nbr_ref, ax_ref, bx_ref):
    i = pl.program_id(0)
    pts_r = pts_ref[...]
    ptsT = ptsT_ref[...]

    sq_r = jnp.zeros((KNN_R, 1), jnp.float32)
    sq_all = jnp.zeros((1, NPTS), jnp.float32)
    for c in range(3):
        sq_r = sq_r + pts_r[:, c:c + 1] * pts_r[:, c:c + 1]
        sq_all = sq_all + ptsT[c:c + 1, :] * ptsT[c:c + 1, :]
    dot = jnp.dot(pts_r.astype(jnp.bfloat16), ptsT.astype(jnp.bfloat16),
                  preferred_element_type=jnp.float32)
    d = (sq_r + sq_all) - 2.0 * dot

    colf = lax.broadcasted_iota(jnp.int32, (KNN_R, NPTS), 1).astype(jnp.float32)
    row_global = (lax.broadcasted_iota(jnp.int32, (KNN_R, NPTS), 0)
                  .astype(jnp.float32) + jnp.float32(i * KNN_R))
    inf = jnp.float32(jnp.inf)
    d = jnp.where(colf == row_global, inf, d)

    bigf = jnp.float32(2**30)
    cols_out = []
    minv = jnp.min(d, axis=1, keepdims=True)
    for k in range(KNN):
        eq = d == minv
        idxf = jnp.min(jnp.where(eq, colf, bigf), axis=1)
        cols_out.append(idxf[:, None])
        if k < KNN - 1:
            d = jnp.where(eq, inf, d)
            minv = jnp.min(d, axis=1, keepdims=True)
    nbr_ref[...] = jnp.concatenate(cols_out, axis=1).astype(jnp.int32)

    ax = jnp.zeros((KNN_R, 64), jnp.float32) + ba_ref[...]
    bx = jnp.zeros((KNN_R, 64), jnp.float32)
    for c in range(3):
        ax = ax + pts_r[:, c:c + 1] * wA_ref[c:c + 1, :]
        bx = bx + pts_r[:, c:c + 1] * wB_ref[c:c + 1, :]
    ax_ref[...] = ax
    bx_ref[...] = bx


def _knn_call(points, pointsT, wA1, wB1, b1a):
    nblk = NPTS // KNN_R
    return pl.pallas_call(
        _knn_body,
        grid=(nblk,),
        in_specs=[
            pl.BlockSpec((KNN_R, 3), lambda i: (i, 0)),
            pl.BlockSpec((3, NPTS), lambda i: (0, 0)),
            pl.BlockSpec((3, 64), lambda i: (0, 0)),
            pl.BlockSpec((3, 64), lambda i: (0, 0)),
            pl.BlockSpec((1, 64), lambda i: (0, 0)),
        ],
        out_specs=[
            pl.BlockSpec((KNN_R, KNN), lambda i: (i, 0)),
            pl.BlockSpec((KNN_R, 64), lambda i: (i, 0)),
            pl.BlockSpec((KNN_R, 64), lambda i: (i, 0)),
        ],
        out_shape=[
            jax.ShapeDtypeStruct((NPTS, KNN), jnp.int32),
            jax.ShapeDtypeStruct((NPTS, 64), jnp.float32),
            jax.ShapeDtypeStruct((NPTS, 64), jnp.float32),
        ],
    )(points, pointsT, wA1, wB1, b1a)


IDX_ROWS = (NPTS * KNN) // 128


def _sc_gather(table, idx2d):
    info = plsc.get_sparse_core_info()
    nc, ns = info.num_cores, info.num_subcores
    nw = nc * ns
    jmax = (IDX_ROWS + nw - 1) // nw
    mesh = plsc.VectorSubcoreMesh(core_axis_name="c", subcore_axis_name="s")

    @functools.partial(
        pl.kernel, mesh=mesh,
        compiler_params=pltpu.CompilerParams(use_tc_tiling_on_sc=False),
        out_type=jax.ShapeDtypeStruct((IDX_ROWS, 128, 64), jnp.float32),
        scratch_types=[
            pltpu.VMEM((128,), jnp.int32),
            pltpu.VMEM((128, 64), jnp.float32),
            pltpu.SemaphoreType.DMA,
        ],
    )
    def gk(table_hbm, idx_hbm, out_hbm, idx_v, rows_v, sem):
        w = lax.axis_index("s") * nc + lax.axis_index("c")

        def body(j, carry):
            row = w + j * nw

            @pl.when(row < IDX_ROWS)
            def _():
                pltpu.sync_copy(idx_hbm.at[row], idx_v)
                pltpu.async_copy(table_hbm.at[idx_v], rows_v, sem).wait()
                pltpu.sync_copy(rows_v, out_hbm.at[row])
            return carry

        lax.fori_loop(0, jmax, body, 0)

    return gk(table, idx2d)


CONV_R = 400


def _convB_A_body(ax_ref, g_ref, wb_ref, bb_ref, wAn_ref, wBn_ref, ban_ref,
                  f_ref, axn_ref, bxn_ref):
    ax = ax_ref[...]
    g = g_ref[...]
    h1 = jax.nn.relu(ax[:, None, :] + g)
    h1f = h1.reshape(CONV_R * KNN, 64)
    h2 = jnp.dot(h1f, wb_ref[...],
                 preferred_element_type=jnp.float32) + bb_ref[...]
    h3 = h2.reshape(CONV_R, KNN, 64)
    f = h3[:, 0, :]
    for k in range(1, KNN):
        f = jnp.maximum(f, h3[:, k, :])
    f_ref[...] = f
    axn_ref[...] = jnp.dot(f, wAn_ref[...],
                           preferred_element_type=jnp.float32) + ban_ref[...]
    bxn_ref[...] = jnp.dot(f, wBn_ref[...], preferred_element_type=jnp.float32)


def _convB_A_call(ax, g3, wb, bb, wAn, wBn, ban):
    nblk = NPTS // CONV_R
    return pl.pallas_call(
        _convB_A_body,
        grid=(nblk,),
        in_specs=[
            pl.BlockSpec((CONV_R, 64), lambda i: (i, 0)),
            pl.BlockSpec((CONV_R, KNN, 64), lambda i: (i, 0, 0)),
            pl.BlockSpec((64, 64), lambda i: (0, 0)),
            pl.BlockSpec((1, 64), lambda i: (0, 0)),
            pl.BlockSpec((64, 64), lambda i: (0, 0)),
            pl.BlockSpec((64, 64), lambda i: (0, 0)),
            pl.BlockSpec((1, 64), lambda i: (0, 0)),
        ],
        out_specs=[
            pl.BlockSpec((CONV_R, 64), lambda i: (i, 0)),
            pl.BlockSpec((CONV_R, 64), lambda i: (i, 0)),
            pl.BlockSpec((CONV_R, 64), lambda i: (i, 0)),
        ],
        out_shape=[
            jax.ShapeDtypeStruct((NPTS, 64), jnp.float32),
            jax.ShapeDtypeStruct((NPTS, 64), jnp.float32),
            jax.ShapeDtypeStruct((NPTS, 64), jnp.float32),
        ],
    )(ax, g3, wb, bb, wAn, wBn, ban)


def _final_body(ax_ref, g_ref, wb_ref, bb_ref, f1_ref, f2_ref,
                w4a_ref, w4b_ref, w4c_ref, b4_ref, w5_ref, b5_ref,
                pts_ref, nT_ref, dist_ref,
                out_ref, cnt_ref, s1_ref):
    i = pl.program_id(0)
    ax = ax_ref[...]
    g = g_ref[...]
    h1 = jax.nn.relu(ax[:, None, :] + g)
    h1f = h1.reshape(CONV_R * KNN, 64)
    h2 = jnp.dot(h1f, wb_ref[...],
                 preferred_element_type=jnp.float32) + bb_ref[...]
    h3 = h2.reshape(CONV_R, KNN, 64)
    f3 = h3[:, 0, :]
    for k in range(1, KNN):
        f3 = jnp.maximum(f3, h3[:, k, :])

    t = (jnp.dot(f1_ref[...], w4a_ref[...], preferred_element_type=jnp.float32)
         + jnp.dot(f2_ref[...], w4b_ref[...], preferred_element_type=jnp.float32)
         + jnp.dot(f3, w4c_ref[...], preferred_element_type=jnp.float32)
         + b4_ref[...])
    t = jax.nn.relu(t)
    res = jnp.dot(t, w5_ref[...], preferred_element_type=jnp.float32) + b5_ref[...]
    pts = pts_ref[...] + res
    out_ref[...] = pts

    nT = nT_ref[...]
    pd = jnp.zeros((CONV_R, NPLANES), jnp.float32) - dist_ref[...]
    for c in range(3):
        pd = pd + pts[:, c:c + 1] * nT[c:c + 1, :]
    m = (jnp.abs(pd) < PLANE_THR).astype(jnp.float32)

    @pl.when(i == 0)
    def _():
        cnt_ref[...] = jnp.zeros_like(cnt_ref)
        s1_ref[...] = jnp.zeros_like(s1_ref)

    cnt_ref[0, :] += jnp.sum(m, axis=0)
    for c in range(3):
        s1_ref[c, :] += jnp.sum(m * pts[:, c:c + 1], axis=0)


def _final_call(ax3, g3, w3b, b3b, f1, f2, w4a, w4b, w4c, b4, w5, b5,
                points, nT, dist):
    nblk = NPTS // CONV_R
    return pl.pallas_call(
        _final_body,
        grid=(nblk,),
        in_specs=[
            pl.BlockSpec((CONV_R, 64), lambda i: (i, 0)),
            pl.BlockSpec((CONV_R, KNN, 64), lambda i: (i, 0, 0)),
            pl.BlockSpec((64, 64), lambda i: (0, 0)),
            pl.BlockSpec((1, 64), lambda i: (0, 0)),
            pl.BlockSpec((CONV_R, 64), lambda i: (i, 0)),
            pl.BlockSpec((CONV_R, 64), lambda i: (i, 0)),
            pl.BlockSpec((64, 256), lambda i: (0, 0)),
            pl.BlockSpec((64, 256), lambda i: (0, 0)),
            pl.BlockSpec((64, 256), lambda i: (0, 0)),
            pl.BlockSpec((1, 256), lambda i: (0, 0)),
            pl.BlockSpec((256, 3), lambda i: (0, 0)),
            pl.BlockSpec((1, 3), lambda i: (0, 0)),
            pl.BlockSpec((CONV_R, 3), lambda i: (i, 0)),
            pl.BlockSpec((3, NPLANES), lambda i: (0, 0)),
            pl.BlockSpec((1, NPLANES), lambda i: (0, 0)),
        ],
        out_specs=[
            pl.BlockSpec((CONV_R, 3), lambda i: (i, 0)),
            pl.BlockSpec((1, NPLANES), lambda i: (0, 0)),
            pl.BlockSpec((3, NPLANES), lambda i: (0, 0)),
        ],
        out_shape=[
            jax.ShapeDtypeStruct((NPTS, 3), jnp.float32),
            jax.ShapeDtypeStruct((1, NPLANES), jnp.float32),
            jax.ShapeDtypeStruct((3, NPLANES), jnp.float32),
        ],
    )(ax3, g3, w3b, b3b, f1, f2, w4a, w4b, w4c, b4, w5, b5, points, nT, dist)


def _jacobi_smallest(cov_rows, nT):
    a = {(0, 0): cov_rows[0], (0, 1): cov_rows[1], (0, 2): cov_rows[2],
         (1, 1): cov_rows[4], (1, 2): cov_rows[5], (2, 2): cov_rows[8]}
    one = jnp.ones_like(cov_rows[0])
    zero = jnp.zeros_like(cov_rows[0])
    v = {(r, c): (one if r == c else zero) for r in range(3) for c in range(3)}

    def A(r, c):
        return a[(r, c)] if r <= c else a[(c, r)]

    for _ in range(6):
        for (p, q) in ((0, 1), (0, 2), (1, 2)):
            apq = A(p, q)
            app = A(p, p)
            aqq = A(q, q)
            tau = (aqq - app) / (2.0 * apq)
            t = jnp.sign(tau) / (jnp.abs(tau) + jnp.sqrt(1.0 + tau * tau))
            t = jnp.where(apq == 0.0, 0.0, t)
            c_ = 1.0 / jnp.sqrt(1.0 + t * t)
            s_ = t * c_
            r = 3 - p - q
            apr, aqr = A(p, r), A(q, r)
            a[(p, p)] = app - t * apq
            a[(q, q)] = aqq + t * apq
            a[(p, q)] = zero
            a[(min(p, r), max(p, r))] = c_ * apr - s_ * aqr
            a[(min(q, r), max(q, r))] = s_ * apr + c_ * aqr
            for i3 in range(3):
                vip, viq = v[(i3, p)], v[(i3, q)]
                v[(i3, p)] = c_ * vip - s_ * viq
                v[(i3, q)] = s_ * vip + c_ * viq

    l0, l1, l2 = a[(0, 0)], a[(1, 1)], a[(2, 2)]
    is0 = (l0 <= l1) & (l0 <= l2)
    is1 = jnp.logical_not(is0) & (l1 <= l2)

    def pick(r):
        return jnp.where(is0, v[(r, 0)], jnp.where(is1, v[(r, 1)], v[(r, 2)]))

    rn = [pick(0), pick(1), pick(2)]
    dotn = rn[0] * nT[0:1, :] + rn[1] * nT[1:2, :] + rn[2] * nT[2:3, :]
    sgn = jnp.where(dotn < 0.0, -1.0, 1.0)
    return [rn[0] * sgn, rn[1] * sgn, rn[2] * sgn]


def _planes_body(pts_ref, nT_ref, dist_ref, cnt_ref, s1_ref,
                 out_ref, cov_s, rn_s, rd_s, val_s):
    i = pl.program_id(0)
    nblk = NPTS // CONV_R
    pts = pts_ref[...]
    nT = nT_ref[...]
    pd = jnp.zeros((CONV_R, NPLANES), jnp.float32) - dist_ref[...]
    for c in range(3):
        pd = pd + pts[:, c:c + 1] * nT[c:c + 1, :]
    m = (jnp.abs(pd) < PLANE_THR).astype(jnp.float32)

    @pl.when(i == 0)
    def _():
        cov_s[...] = jnp.zeros_like(cov_s)

    @pl.when(i < nblk)
    def _():
        cnt = jnp.maximum(cnt_ref[...], 1.0)
        cen = [(pts[:, c:c + 1] - s1_ref[c:c + 1, :] / cnt) * m
               for c in range(3)]
        j = 0
        for aa in range(3):
            for bb in range(3):
                cov_s[j, :] += jnp.sum(cen[aa] * cen[bb], axis=0)
                j += 1

    @pl.when(i == nblk)
    def _():
        cnt = jnp.maximum(cnt_ref[...], 1.0)
        ct = [s1_ref[c:c + 1, :] / cnt for c in range(3)]
        rn = _jacobi_smallest([cov_s[j:j + 1, :] for j in range(9)],
                              nT_ref[...])
        rd = ct[0] * rn[0] + ct[1] * rn[1] + ct[2] * rn[2]
        for c in range(3):
            rn_s[c:c + 1, :] = rn[c]
        rd_s[...] = rd
        val_s[...] = (cnt_ref[...] >= 3.0).astype(jnp.float32)

    @pl.when(i >= nblk)
    def _():
        rnT = rn_s[...]
        rd = rd_s[...]
        valid = val_s[...]
        px = pts[:, 0]
        py = pts[:, 1]
        pz = pts[:, 2]
        for p in range(NPLANES):
            coef = valid[0, p] * m[:, p]
            dot = px * rnT[0, p] + py * rnT[1, p] + pz * rnT[2, p]
            scale = coef * (dot - rd[0, p])
            px = px - scale * rnT[0, p]
            py = py - scale * rnT[1, p]
            pz = pz - scale * rnT[2, p]
        out_ref[...] = jnp.concatenate(
            [px[:, None], py[:, None], pz[:, None]], axis=1)


def _planes_call(pts, nT, dist, cnt2d, s1):
    nblk = NPTS // CONV_R
    return pl.pallas_call(
        _planes_body,
        grid=(2 * nblk,),
        in_specs=[
            pl.BlockSpec((CONV_R, 3), lambda i: (lax.rem(i, nblk), 0)),
            pl.BlockSpec((3, NPLANES), lambda i: (0, 0)),
            pl.BlockSpec((1, NPLANES), lambda i: (0, 0)),
            pl.BlockSpec((1, NPLANES), lambda i: (0, 0)),
            pl.BlockSpec((3, NPLANES), lambda i: (0, 0)),
        ],
        out_specs=pl.BlockSpec((CONV_R, 3),
                               lambda i: (jnp.maximum(i - nblk, 0), 0)),
        out_shape=jax.ShapeDtypeStruct((NPTS, 3), jnp.float32),
        scratch_shapes=[
            pltpu.VMEM((9, NPLANES), jnp.float32),
            pltpu.VMEM((3, NPLANES), jnp.float32),
            pltpu.VMEM((1, NPLANES), jnp.float32),
            pltpu.VMEM((1, NPLANES), jnp.float32),
        ],
    )(pts, nT, dist, cnt2d, s1)


def kernel(points, normals, distances, w1a, b1a, w1b, b1b, w2a, b2a, w2b, b2b,
           w3a, b3a, w3b, b3b, w4, b4, w5, b5):
    f32 = jnp.float32
    pointsT = points.T
    wA1, wB1 = w1a[:3] - w1a[3:], w1a[3:]
    wA2, wB2 = w2a[:64] - w2a[64:], w2a[64:]
    wA3, wB3 = w3a[:64] - w3a[64:], w3a[64:]

    nbrs, ax1, bx1 = _knn_call(points, pointsT, wA1, wB1, b1a[None, :])
    idx2d = nbrs.reshape(IDX_ROWS, 128)

    g1 = _sc_gather(bx1, idx2d).reshape(NPTS, KNN, 64)
    f1, ax2, bx2 = _convB_A_call(ax1, g1, w1b, b1b[None, :],
                                 wA2, wB2, b2a[None, :])
    g2 = _sc_gather(bx2, idx2d).reshape(NPTS, KNN, 64)
    f2, ax3, bx3 = _convB_A_call(ax2, g2, w2b, b2b[None, :],
                                 wA3, wB3, b3a[None, :])
    g3 = _sc_gather(bx3, idx2d).reshape(NPTS, KNN, 64)

    nT = normals.T.astype(f32)
    dist = distances[None, :].astype(f32)
    pts, cnt2d, s1 = _final_call(
        ax3, g3, w3b, b3b[None, :], f1, f2,
        w4[0:64], w4[64:128], w4[128:192], b4[None, :], w5, b5[None, :],
        points, nT, dist)

    return _planes_call(pts, nT, dist, cnt2d, s1)


# speedup vs baseline: 6.5960x; 1.0698x over previous
import functools

import jax
import jax.numpy as jnp
from jax import lax
from jax.experimental import pallas as pl
from jax.experimental.pallas import tpu as pltpu
from jax.experimental.pallas import tpu_sc as plsc

NPTS = 10000
KNN = 16
NPLANES = 8
PLANE_THR = 0.05


KNN_R = 80


def _knn_body(pts_ref, ptsT_ref, wA_ref, wB_ref, ba_ref,
              nbr_ref, ax_ref, bx_ref):
    i = pl.program_id(0)
    pts_r = pts_ref[...]
    ptsT = ptsT_ref[...]

    sq_r = jnp.zeros((KNN_R, 1), jnp.float32)
    sq_all = jnp.zeros((1, NPTS), jnp.float32)
    for c in range(3):
        sq_r = sq_r + pts_r[:, c:c + 1] * pts_r[:, c:c + 1]
        sq_all = sq_all + ptsT[c:c + 1, :] * ptsT[c:c + 1, :]
    dot = jnp.dot(pts_r.astype(jnp.bfloat16), ptsT.astype(jnp.bfloat16),
                  preferred_element_type=jnp.float32)
    d = (sq_r + sq_all) - 2.0 * dot

    colf = lax.broadcasted_iota(jnp.int32, (KNN_R, NPTS), 1).astype(jnp.float32)
    row_global = (lax.broadcasted_iota(jnp.int32, (KNN_R, NPTS), 0)
                  .astype(jnp.float32) + jnp.float32(i * KNN_R))
    inf = jnp.float32(jnp.inf)
    d = jnp.where(colf == row_global, inf, d)

    bigf = jnp.float32(2**30)
    cols_out = []
    minv = jnp.min(d, axis=1, keepdims=True)
    for k in range(KNN):
        eq = d == minv
        idxf = jnp.min(jnp.where(eq, colf, bigf), axis=1)
        cols_out.append(idxf[:, None])
        if k < KNN - 1:
            d = jnp.where(eq, inf, d)
            minv = jnp.min(d, axis=1, keepdims=True)
    nbr_ref[...] = jnp.concatenate(cols_out, axis=1).astype(jnp.int32)

    ax = jnp.zeros((KNN_R, 64), jnp.float32) + ba_ref[...]
    bx = jnp.zeros((KNN_R, 64), jnp.float32)
    for c in range(3):
        ax = ax + pts_r[:, c:c + 1] * wA_ref[c:c + 1, :]
        bx = bx + pts_r[:, c:c + 1] * wB_ref[c:c + 1, :]
    ax_ref[...] = ax
    bx_ref[...] = bx


def _knn_call(points, pointsT, wA1, wB1, b1a):
    nblk = NPTS // KNN_R
    return pl.pallas_call(
        _knn_body,
        grid=(nblk,),
        in_specs=[
            pl.BlockSpec((KNN_R, 3), lambda i: (i, 0)),
            pl.BlockSpec((3, NPTS), lambda i: (0, 0)),
            pl.BlockSpec((3, 64), lambda i: (0, 0)),
            pl.BlockSpec((3, 64), lambda i: (0, 0)),
            pl.BlockSpec((1, 64), lambda i: (0, 0)),
        ],
        out_specs=[
            pl.BlockSpec((KNN_R, KNN), lambda i: (i, 0)),
            pl.BlockSpec((KNN_R, 64), lambda i: (i, 0)),
            pl.BlockSpec((KNN_R, 64), lambda i: (i, 0)),
        ],
        out_shape=[
            jax.ShapeDtypeStruct((NPTS, KNN), jnp.int32),
            jax.ShapeDtypeStruct((NPTS, 64), jnp.float32),
            jax.ShapeDtypeStruct((NPTS, 64), jnp.float32),
        ],
    )(points, pointsT, wA1, wB1, b1a)


IDX_ROWS = (NPTS * KNN) // 128


def _sc_gather(table, idx2d):
    info = plsc.get_sparse_core_info()
    nc, ns = info.num_cores, info.num_subcores
    nw = nc * ns
    jmax = (IDX_ROWS + nw - 1) // nw
    mesh = plsc.VectorSubcoreMesh(core_axis_name="c", subcore_axis_name="s")

    @functools.partial(
        pl.kernel, mesh=mesh,
        compiler_params=pltpu.CompilerParams(use_tc_tiling_on_sc=False),
        out_type=jax.ShapeDtypeStruct((IDX_ROWS, 128, 64), jnp.float32),
        scratch_types=[
            pltpu.VMEM((2, 128), jnp.int32),
            pltpu.VMEM((2, 128, 64), jnp.float32),
            pltpu.SemaphoreType.DMA,
            pltpu.SemaphoreType.DMA,
            pltpu.SemaphoreType.DMA,
            pltpu.SemaphoreType.DMA,
        ],
    )
    def gk(table_hbm, idx_hbm, out_hbm, idx_v, rows_v, g0, g1, o0, o1):
        w = lax.axis_index("s") * nc + lax.axis_index("c")
        gsem = (g0, g1)
        osem = (o0, o1)

        def start_gather(j, b):
            pltpu.sync_copy(idx_hbm.at[j * nw + w], idx_v.at[b])
            pltpu.async_copy(table_hbm.at[idx_v.at[b]], rows_v.at[b], gsem[b])

        def wait_gather(b):
            pltpu.make_async_copy(table_hbm.at[idx_v.at[b]], rows_v.at[b],
                                  gsem[b]).wait()

        def start_out(j, b):
            pltpu.async_copy(rows_v.at[b], out_hbm.at[j * nw + w], osem[b])

        def wait_out(j, b):
            pltpu.make_async_copy(rows_v.at[b], out_hbm.at[j * nw + w],
                                  osem[b]).wait()

        start_gather(0, 0)

        def body(g, carry):
            for b in range(2):
                j = g * 2 + b
                jn = j + 1
                bn = 1 - b

                @pl.when(jn * nw + w < IDX_ROWS)
                def _():
                    @pl.when(j >= 1)
                    def _():
                        wait_out(j - 1, bn)
                    start_gather(jn, bn)

                @pl.when(j * nw + w < IDX_ROWS)
                def _():
                    wait_gather(b)
                    start_out(j, b)
            return carry

        lax.fori_loop(0, jmax // 2, body, 0)
        wait_out(jmax - 2, 0)

        @pl.when((jmax - 1) * nw + w < IDX_ROWS)
        def _():
            wait_out(jmax - 1, 1)

    return gk(table, idx2d)


CONV_R = 400


def _convB_A_body(ax_ref, g_ref, wb_ref, bb_ref, wAn_ref, wBn_ref, ban_ref,
                  f_ref, axn_ref, bxn_ref):
    ax = ax_ref[...]
    g = g_ref[...]
    h1 = jax.nn.relu(ax[:, None, :] + g)
    h1f = h1.reshape(CONV_R * KNN, 64)
    h2 = jnp.dot(h1f, wb_ref[...],
                 preferred_element_type=jnp.float32) + bb_ref[...]
    f = jnp.max(h2.reshape(CONV_R, KNN, 64), axis=1)
    f_ref[...] = f
    axn_ref[...] = jnp.dot(f, wAn_ref[...],
                           preferred_element_type=jnp.float32) + ban_ref[...]
    bxn_ref[...] = jnp.dot(f, wBn_ref[...], preferred_element_type=jnp.float32)


def _convB_A_call(ax, g3, wb, bb, wAn, wBn, ban):
    nblk = NPTS // CONV_R
    return pl.pallas_call(
        _convB_A_body,
        grid=(nblk,),
        in_specs=[
            pl.BlockSpec((CONV_R, 64), lambda i: (i, 0)),
            pl.BlockSpec((CONV_R, KNN, 64), lambda i: (i, 0, 0)),
            pl.BlockSpec((64, 64), lambda i: (0, 0)),
            pl.BlockSpec((1, 64), lambda i: (0, 0)),
            pl.BlockSpec((64, 64), lambda i: (0, 0)),
            pl.BlockSpec((64, 64), lambda i: (0, 0)),
            pl.BlockSpec((1, 64), lambda i: (0, 0)),
        ],
        out_specs=[
            pl.BlockSpec((CONV_R, 64), lambda i: (i, 0)),
            pl.BlockSpec((CONV_R, 64), lambda i: (i, 0)),
            pl.BlockSpec((CONV_R, 64), lambda i: (i, 0)),
        ],
        out_shape=[
            jax.ShapeDtypeStruct((NPTS, 64), jnp.float32),
            jax.ShapeDtypeStruct((NPTS, 64), jnp.float32),
            jax.ShapeDtypeStruct((NPTS, 64), jnp.float32),
        ],
    )(ax, g3, wb, bb, wAn, wBn, ban)


def _final_body(ax_ref, g_ref, wb_ref, bb_ref, f1_ref, f2_ref,
                w4a_ref, w4b_ref, w4c_ref, b4_ref, w5_ref, b5_ref,
                pts_ref, nT_ref, dist_ref,
                out_ref, cnt_ref, s1_ref):
    i = pl.program_id(0)
    ax = ax_ref[...]
    g = g_ref[...]
    h1 = jax.nn.relu(ax[:, None, :] + g)
    h1f = h1.reshape(CONV_R * KNN, 64)
    h2 = jnp.dot(h1f, wb_ref[...],
                 preferred_element_type=jnp.float32) + bb_ref[...]
    f3 = jnp.max(h2.reshape(CONV_R, KNN, 64), axis=1)

    t = (jnp.dot(f1_ref[...], w4a_ref[...], preferred_element_type=jnp.float32)
         + jnp.dot(f2_ref[...], w4b_ref[...], preferred_element_type=jnp.float32)
         + jnp.dot(f3, w4c_ref[...], preferred_element_type=jnp.float32)
         + b4_ref[...])
    t = jax.nn.relu(t)
    res = jnp.dot(t, w5_ref[...], preferred_element_type=jnp.float32) + b5_ref[...]
    pts = pts_ref[...] + res
    out_ref[...] = pts

    nT = nT_ref[...]
    pd = jnp.zeros((CONV_R, NPLANES), jnp.float32) - dist_ref[...]
    for c in range(3):
        pd = pd + pts[:, c:c + 1] * nT[c:c + 1, :]
    m = (jnp.abs(pd) < PLANE_THR).astype(jnp.float32)

    @pl.when(i == 0)
    def _():
        cnt_ref[...] = jnp.zeros_like(cnt_ref)
        s1_ref[...] = jnp.zeros_like(s1_ref)

    cnt_ref[0, :] += jnp.sum(m, axis=0)
    for c in range(3):
        s1_ref[c, :] += jnp.sum(m * pts[:, c:c + 1], axis=0)


def _final_call(ax3, g3, w3b, b3b, f1, f2, w4a, w4b, w4c, b4, w5, b5,
                points, nT, dist):
    nblk = NPTS // CONV_R
    return pl.pallas_call(
        _final_body,
        grid=(nblk,),
        in_specs=[
            pl.BlockSpec((CONV_R, 64), lambda i: (i, 0)),
            pl.BlockSpec((CONV_R, KNN, 64), lambda i: (i, 0, 0)),
            pl.BlockSpec((64, 64), lambda i: (0, 0)),
            pl.BlockSpec((1, 64), lambda i: (0, 0)),
            pl.BlockSpec((CONV_R, 64), lambda i: (i, 0)),
            pl.BlockSpec((CONV_R, 64), lambda i: (i, 0)),
            pl.BlockSpec((64, 256), lambda i: (0, 0)),
            pl.BlockSpec((64, 256), lambda i: (0, 0)),
            pl.BlockSpec((64, 256), lambda i: (0, 0)),
            pl.BlockSpec((1, 256), lambda i: (0, 0)),
            pl.BlockSpec((256, 3), lambda i: (0, 0)),
            pl.BlockSpec((1, 3), lambda i: (0, 0)),
            pl.BlockSpec((CONV_R, 3), lambda i: (i, 0)),
            pl.BlockSpec((3, NPLANES), lambda i: (0, 0)),
            pl.BlockSpec((1, NPLANES), lambda i: (0, 0)),
        ],
        out_specs=[
            pl.BlockSpec((CONV_R, 3), lambda i: (i, 0)),
            pl.BlockSpec((1, NPLANES), lambda i: (0, 0)),
            pl.BlockSpec((3, NPLANES), lambda i: (0, 0)),
        ],
        out_shape=[
            jax.ShapeDtypeStruct((NPTS, 3), jnp.float32),
            jax.ShapeDtypeStruct((1, NPLANES), jnp.float32),
            jax.ShapeDtypeStruct((3, NPLANES), jnp.float32),
        ],
    )(ax3, g3, w3b, b3b, f1, f2, w4a, w4b, w4c, b4, w5, b5, points, nT, dist)


def _jacobi_smallest(cov_rows, nT):
    a = {(0, 0): cov_rows[0], (0, 1): cov_rows[1], (0, 2): cov_rows[2],
         (1, 1): cov_rows[4], (1, 2): cov_rows[5], (2, 2): cov_rows[8]}
    one = jnp.ones_like(cov_rows[0])
    zero = jnp.zeros_like(cov_rows[0])
    v = {(r, c): (one if r == c else zero) for r in range(3) for c in range(3)}

    def A(r, c):
        return a[(r, c)] if r <= c else a[(c, r)]

    for _ in range(6):
        for (p, q) in ((0, 1), (0, 2), (1, 2)):
            apq = A(p, q)
            app = A(p, p)
            aqq = A(q, q)
            tau = (aqq - app) / (2.0 * apq)
            t = jnp.sign(tau) / (jnp.abs(tau) + jnp.sqrt(1.0 + tau * tau))
            t = jnp.where(apq == 0.0, 0.0, t)
            c_ = 1.0 / jnp.sqrt(1.0 + t * t)
            s_ = t * c_
            r = 3 - p - q
            apr, aqr = A(p, r), A(q, r)
            a[(p, p)] = app - t * apq
            a[(q, q)] = aqq + t * apq
            a[(p, q)] = zero
            a[(min(p, r), max(p, r))] = c_ * apr - s_ * aqr
            a[(min(q, r), max(q, r))] = s_ * apr + c_ * aqr
            for i3 in range(3):
                vip, viq = v[(i3, p)], v[(i3, q)]
                v[(i3, p)] = c_ * vip - s_ * viq
                v[(i3, q)] = s_ * vip + c_ * viq

    l0, l1, l2 = a[(0, 0)], a[(1, 1)], a[(2, 2)]
    is0 = (l0 <= l1) & (l0 <= l2)
    is1 = jnp.logical_not(is0) & (l1 <= l2)

    def pick(r):
        return jnp.where(is0, v[(r, 0)], jnp.where(is1, v[(r, 1)], v[(r, 2)]))

    rn = [pick(0), pick(1), pick(2)]
    dotn = rn[0] * nT[0:1, :] + rn[1] * nT[1:2, :] + rn[2] * nT[2:3, :]
    sgn = jnp.where(dotn < 0.0, -1.0, 1.0)
    return [rn[0] * sgn, rn[1] * sgn, rn[2] * sgn]


def _planes_body(pts_ref, nT_ref, dist_ref, cnt_ref, s1_ref,
                 out_ref, cov_s, rn_s, rd_s, val_s):
    i = pl.program_id(0)
    nblk = NPTS // CONV_R
    pts = pts_ref[...]
    nT = nT_ref[...]
    pd = jnp.zeros((CONV_R, NPLANES), jnp.float32) - dist_ref[...]
    for c in range(3):
        pd = pd + pts[:, c:c + 1] * nT[c:c + 1, :]
    m = (jnp.abs(pd) < PLANE_THR).astype(jnp.float32)

    @pl.when(i == 0)
    def _():
        cov_s[...] = jnp.zeros_like(cov_s)

    @pl.when(i < nblk)
    def _():
        cnt = jnp.maximum(cnt_ref[...], 1.0)
        cen = [(pts[:, c:c + 1] - s1_ref[c:c + 1, :] / cnt) * m
               for c in range(3)]
        j = 0
        for aa in range(3):
            for bb in range(3):
                cov_s[j, :] += jnp.sum(cen[aa] * cen[bb], axis=0)
                j += 1

    @pl.when(i == nblk)
    def _():
        cnt = jnp.maximum(cnt_ref[...], 1.0)
        ct = [s1_ref[c:c + 1, :] / cnt for c in range(3)]
        rn = _jacobi_smallest([cov_s[j:j + 1, :] for j in range(9)],
                              nT_ref[...])
        rd = ct[0] * rn[0] + ct[1] * rn[1] + ct[2] * rn[2]
        for c in range(3):
            rn_s[c:c + 1, :] = rn[c]
        rd_s[...] = rd
        val_s[...] = (cnt_ref[...] >= 3.0).astype(jnp.float32)

    @pl.when(i >= nblk)
    def _():
        rnT = rn_s[...]
        rd = rd_s[...]
        valid = val_s[...]
        px = pts[:, 0]
        py = pts[:, 1]
        pz = pts[:, 2]
        for p in range(NPLANES):
            coef = valid[0, p] * m[:, p]
            dot = px * rnT[0, p] + py * rnT[1, p] + pz * rnT[2, p]
            scale = coef * (dot - rd[0, p])
            px = px - scale * rnT[0, p]
            py = py - scale * rnT[1, p]
            pz = pz - scale * rnT[2, p]
        out_ref[...] = jnp.concatenate(
            [px[:, None], py[:, None], pz[:, None]], axis=1)


def _planes_call(pts, nT, dist, cnt2d, s1):
    nblk = NPTS // CONV_R
    return pl.pallas_call(
        _planes_body,
        grid=(2 * nblk,),
        in_specs=[
            pl.BlockSpec((CONV_R, 3), lambda i: (lax.rem(i, nblk), 0)),
            pl.BlockSpec((3, NPLANES), lambda i: (0, 0)),
            pl.BlockSpec((1, NPLANES), lambda i: (0, 0)),
            pl.BlockSpec((1, NPLANES), lambda i: (0, 0)),
            pl.BlockSpec((3, NPLANES), lambda i: (0, 0)),
        ],
        out_specs=pl.BlockSpec((CONV_R, 3),
                               lambda i: (jnp.maximum(i - nblk, 0), 0)),
        out_shape=jax.ShapeDtypeStruct((NPTS, 3), jnp.float32),
        scratch_shapes=[
            pltpu.VMEM((9, NPLANES), jnp.float32),
            pltpu.VMEM((3, NPLANES), jnp.float32),
            pltpu.VMEM((1, NPLANES), jnp.float32),
            pltpu.VMEM((1, NPLANES), jnp.float32),
        ],
    )(pts, nT, dist, cnt2d, s1)


def kernel(points, normals, distances, w1a, b1a, w1b, b1b, w2a, b2a, w2b, b2b,
           w3a, b3a, w3b, b3b, w4, b4, w5, b5):
    f32 = jnp.float32
    pointsT = points.T
    wA1, wB1 = w1a[:3] - w1a[3:], w1a[3:]
    wA2, wB2 = w2a[:64] - w2a[64:], w2a[64:]
    wA3, wB3 = w3a[:64] - w3a[64:], w3a[64:]

    nbrs, ax1, bx1 = _knn_call(points, pointsT, wA1, wB1, b1a[None, :])
    idx2d = nbrs.reshape(IDX_ROWS, 128)

    g1 = _sc_gather(bx1, idx2d).reshape(NPTS, KNN, 64)
    f1, ax2, bx2 = _convB_A_call(ax1, g1, w1b, b1b[None, :],
                                 wA2, wB2, b2a[None, :])
    g2 = _sc_gather(bx2, idx2d).reshape(NPTS, KNN, 64)
    f2, ax3, bx3 = _convB_A_call(ax2, g2, w2b, b2b[None, :],
                                 wA3, wB3, b3a[None, :])
    g3 = _sc_gather(bx3, idx2d).reshape(NPTS, KNN, 64)

    nT = normals.T.astype(f32)
    dist = distances[None, :].astype(f32)
    pts, cnt2d, s1 = _final_call(
        ax3, g3, w3b, b3b[None, :], f1, f2,
        w4[0:64], w4[64:128], w4[128:192], b4[None, :], w5, b5[None, :],
        points, nT, dist)

    return _planes_call(pts, nT, dist, cnt2d, s1)


# speedup vs baseline: 6.7798x; 1.0279x over previous
import functools

import jax
import jax.numpy as jnp
from jax import lax
from jax.experimental import pallas as pl
from jax.experimental.pallas import tpu as pltpu
from jax.experimental.pallas import tpu_sc as plsc

NPTS = 10000
KNN = 16
NPLANES = 8
PLANE_THR = 0.05


KNN_R = 80


def _knn_body(pts_ref, ptsT_ref, wA_ref, wB_ref, ba_ref,
              nbr_ref, ax_ref, bx_ref):
    i = pl.program_id(0)
    pts_r = pts_ref[...]
    ptsT = ptsT_ref[...]

    sq_r = jnp.zeros((KNN_R, 1), jnp.float32)
    sq_all = jnp.zeros((1, NPTS), jnp.float32)
    for c in range(3):
        sq_r = sq_r + pts_r[:, c:c + 1] * pts_r[:, c:c + 1]
        sq_all = sq_all + ptsT[c:c + 1, :] * ptsT[c:c + 1, :]
    dot = jnp.dot(pts_r.astype(jnp.bfloat16), ptsT.astype(jnp.bfloat16),
                  preferred_element_type=jnp.float32)
    d = (sq_r + sq_all) - 2.0 * dot

    colf = lax.broadcasted_iota(jnp.int32, (KNN_R, NPTS), 1).astype(jnp.float32)
    row_global = (lax.broadcasted_iota(jnp.int32, (KNN_R, NPTS), 0)
                  .astype(jnp.float32) + jnp.float32(i * KNN_R))
    inf = jnp.float32(jnp.inf)
    d = jnp.where(colf == row_global, inf, d)

    bigf = jnp.float32(2**30)
    cols_out = []
    minv = jnp.min(d, axis=1, keepdims=True)
    for k in range(KNN):
        eq = d == minv
        idxf = jnp.min(jnp.where(eq, colf, bigf), axis=1)
        cols_out.append(idxf[:, None])
        if k < KNN - 1:
            d = jnp.where(eq, inf, d)
            minv = jnp.min(d, axis=1, keepdims=True)
    nbr_ref[...] = jnp.concatenate(cols_out, axis=1).astype(jnp.int32)

    ax = jnp.zeros((KNN_R, 64), jnp.float32) + ba_ref[...]
    bx = jnp.zeros((KNN_R, 64), jnp.float32)
    for c in range(3):
        ax = ax + pts_r[:, c:c + 1] * wA_ref[c:c + 1, :]
        bx = bx + pts_r[:, c:c + 1] * wB_ref[c:c + 1, :]
    ax_ref[...] = ax
    bx_ref[...] = bx


def _knn_call(points, pointsT, wA1, wB1, b1a):
    nblk = NPTS // KNN_R
    return pl.pallas_call(
        _knn_body,
        grid=(nblk,),
        in_specs=[
            pl.BlockSpec((KNN_R, 3), lambda i: (i, 0)),
            pl.BlockSpec((3, NPTS), lambda i: (0, 0)),
            pl.BlockSpec((3, 64), lambda i: (0, 0)),
            pl.BlockSpec((3, 64), lambda i: (0, 0)),
            pl.BlockSpec((1, 64), lambda i: (0, 0)),
        ],
        out_specs=[
            pl.BlockSpec((KNN_R, KNN), lambda i: (i, 0)),
            pl.BlockSpec((KNN_R, 64), lambda i: (i, 0)),
            pl.BlockSpec((KNN_R, 64), lambda i: (i, 0)),
        ],
        out_shape=[
            jax.ShapeDtypeStruct((NPTS, KNN), jnp.int32),
            jax.ShapeDtypeStruct((NPTS, 64), jnp.float32),
            jax.ShapeDtypeStruct((NPTS, 64), jnp.float32),
        ],
    )(points, pointsT, wA1, wB1, b1a)


IDX_ROWS = (NPTS * KNN) // 128


def _sc_gather(table, idx2d):
    info = plsc.get_sparse_core_info()
    nc, ns = info.num_cores, info.num_subcores
    nw = nc * ns
    jmax = (IDX_ROWS + nw - 1) // nw
    mesh = plsc.VectorSubcoreMesh(core_axis_name="c", subcore_axis_name="s")

    @functools.partial(
        pl.kernel, mesh=mesh,
        compiler_params=pltpu.CompilerParams(use_tc_tiling_on_sc=False),
        out_type=jax.ShapeDtypeStruct((IDX_ROWS, 128, 64), jnp.float32),
        scratch_types=[
            pltpu.VMEM((2, 128), jnp.int32),
            pltpu.VMEM((2, 128, 64), jnp.float32),
            pltpu.SemaphoreType.DMA,
            pltpu.SemaphoreType.DMA,
            pltpu.SemaphoreType.DMA,
            pltpu.SemaphoreType.DMA,
        ],
    )
    def gk(table_hbm, idx_hbm, out_hbm, idx_v, rows_v, g0, g1, o0, o1):
        w = lax.axis_index("s") * nc + lax.axis_index("c")
        gsem = (g0, g1)
        osem = (o0, o1)

        def start_gather(j, b):
            pltpu.sync_copy(idx_hbm.at[j * nw + w], idx_v.at[b])
            pltpu.async_copy(table_hbm.at[idx_v.at[b]], rows_v.at[b], gsem[b])

        def wait_gather(b):
            pltpu.make_async_copy(table_hbm.at[idx_v.at[b]], rows_v.at[b],
                                  gsem[b]).wait()

        def start_out(j, b):
            pltpu.async_copy(rows_v.at[b], out_hbm.at[j * nw + w], osem[b])

        def wait_out(j, b):
            pltpu.make_async_copy(rows_v.at[b], out_hbm.at[j * nw + w],
                                  osem[b]).wait()

        start_gather(0, 0)

        def body(g, carry):
            for b in range(2):
                j = g * 2 + b
                jn = j + 1
                bn = 1 - b

                @pl.when(jn * nw + w < IDX_ROWS)
                def _():
                    @pl.when(j >= 1)
                    def _():
                        wait_out(j - 1, bn)
                    start_gather(jn, bn)

                @pl.when(j * nw + w < IDX_ROWS)
                def _():
                    wait_gather(b)
                    start_out(j, b)
            return carry

        lax.fori_loop(0, jmax // 2, body, 0)
        wait_out(jmax - 2, 0)

        @pl.when((jmax - 1) * nw + w < IDX_ROWS)
        def _():
            wait_out(jmax - 1, 1)

    return gk(table, idx2d)


CONV_R = 400


def _convB_A_body(ax_ref, g_ref, wb_ref, bb_ref, wAn_ref, wBn_ref, ban_ref,
                  f_ref, axn_ref, bxn_ref):
    ax = ax_ref[...]
    g = g_ref[...]
    h1 = jax.nn.relu(ax[:, None, :] + g)
    h1f = h1.reshape(CONV_R * KNN, 64)
    h2 = jnp.dot(h1f, wb_ref[...],
                 preferred_element_type=jnp.float32) + bb_ref[...]
    f = jnp.max(h2.reshape(CONV_R, KNN, 64), axis=1)
    f_ref[...] = f
    axn_ref[...] = jnp.dot(f, wAn_ref[...],
                           preferred_element_type=jnp.float32) + ban_ref[...]
    bxn_ref[...] = jnp.dot(f, wBn_ref[...], preferred_element_type=jnp.float32)


def _convB_A_call(ax, g3, wb, bb, wAn, wBn, ban):
    nblk = NPTS // CONV_R
    return pl.pallas_call(
        _convB_A_body,
        grid=(nblk,),
        in_specs=[
            pl.BlockSpec((CONV_R, 64), lambda i: (i, 0)),
            pl.BlockSpec((CONV_R, KNN, 64), lambda i: (i, 0, 0)),
            pl.BlockSpec((64, 64), lambda i: (0, 0)),
            pl.BlockSpec((1, 64), lambda i: (0, 0)),
            pl.BlockSpec((64, 64), lambda i: (0, 0)),
            pl.BlockSpec((64, 64), lambda i: (0, 0)),
            pl.BlockSpec((1, 64), lambda i: (0, 0)),
        ],
        out_specs=[
            pl.BlockSpec((CONV_R, 64), lambda i: (i, 0)),
            pl.BlockSpec((CONV_R, 64), lambda i: (i, 0)),
            pl.BlockSpec((CONV_R, 64), lambda i: (i, 0)),
        ],
        out_shape=[
            jax.ShapeDtypeStruct((NPTS, 64), jnp.float32),
            jax.ShapeDtypeStruct((NPTS, 64), jnp.float32),
            jax.ShapeDtypeStruct((NPTS, 64), jnp.float32),
        ],
    )(ax, g3, wb, bb, wAn, wBn, ban)


def _final_body(ax_ref, g_ref, wb_ref, bb_ref, f1_ref, f2_ref,
                w4a_ref, w4b_ref, w4c_ref, b4_ref, w5_ref, b5_ref,
                pts_ref, nT_ref, dist_ref,
                out_ref, cnt_ref, s1_ref):
    i = pl.program_id(0)
    ax = ax_ref[...]
    g = g_ref[...]
    h1 = jax.nn.relu(ax[:, None, :] + g)
    h1f = h1.reshape(CONV_R * KNN, 64)
    h2 = jnp.dot(h1f, wb_ref[...],
                 preferred_element_type=jnp.float32) + bb_ref[...]
    f3 = jnp.max(h2.reshape(CONV_R, KNN, 64), axis=1)

    t = (jnp.dot(f1_ref[...], w4a_ref[...], preferred_element_type=jnp.float32)
         + jnp.dot(f2_ref[...], w4b_ref[...], preferred_element_type=jnp.float32)
         + jnp.dot(f3, w4c_ref[...], preferred_element_type=jnp.float32)
         + b4_ref[...])
    t = jax.nn.relu(t)
    res = jnp.dot(t, w5_ref[...], preferred_element_type=jnp.float32) + b5_ref[...]
    pts = pts_ref[...] + res
    out_ref[...] = pts

    nT = nT_ref[...]
    pd = jnp.zeros((CONV_R, NPLANES), jnp.float32) - dist_ref[...]
    for c in range(3):
        pd = pd + pts[:, c:c + 1] * nT[c:c + 1, :]
    m = (jnp.abs(pd) < PLANE_THR).astype(jnp.float32)

    @pl.when(i == 0)
    def _():
        cnt_ref[...] = jnp.zeros_like(cnt_ref)
        s1_ref[...] = jnp.zeros_like(s1_ref)

    cnt_ref[0, :] += jnp.sum(m, axis=0)
    for c in range(3):
        s1_ref[c, :] += jnp.sum(m * pts[:, c:c + 1], axis=0)


def _final_call(ax3, g3, w3b, b3b, f1, f2, w4a, w4b, w4c, b4, w5, b5,
                points, nT, dist):
    nblk = NPTS // CONV_R
    return pl.pallas_call(
        _final_body,
        grid=(nblk,),
        in_specs=[
            pl.BlockSpec((CONV_R, 64), lambda i: (i, 0)),
            pl.BlockSpec((CONV_R, KNN, 64), lambda i: (i, 0, 0)),
            pl.BlockSpec((64, 64), lambda i: (0, 0)),
            pl.BlockSpec((1, 64), lambda i: (0, 0)),
            pl.BlockSpec((CONV_R, 64), lambda i: (i, 0)),
            pl.BlockSpec((CONV_R, 64), lambda i: (i, 0)),
            pl.BlockSpec((64, 256), lambda i: (0, 0)),
            pl.BlockSpec((64, 256), lambda i: (0, 0)),
            pl.BlockSpec((64, 256), lambda i: (0, 0)),
            pl.BlockSpec((1, 256), lambda i: (0, 0)),
            pl.BlockSpec((256, 3), lambda i: (0, 0)),
            pl.BlockSpec((1, 3), lambda i: (0, 0)),
            pl.BlockSpec((CONV_R, 3), lambda i: (i, 0)),
            pl.BlockSpec((3, NPLANES), lambda i: (0, 0)),
            pl.BlockSpec((1, NPLANES), lambda i: (0, 0)),
        ],
        out_specs=[
            pl.BlockSpec((CONV_R, 3), lambda i: (i, 0)),
            pl.BlockSpec((1, NPLANES), lambda i: (0, 0)),
            pl.BlockSpec((3, NPLANES), lambda i: (0, 0)),
        ],
        out_shape=[
            jax.ShapeDtypeStruct((NPTS, 3), jnp.float32),
            jax.ShapeDtypeStruct((1, NPLANES), jnp.float32),
            jax.ShapeDtypeStruct((3, NPLANES), jnp.float32),
        ],
    )(ax3, g3, w3b, b3b, f1, f2, w4a, w4b, w4c, b4, w5, b5, points, nT, dist)


def _jacobi_smallest(cov_rows, nT):
    a = {(0, 0): cov_rows[0], (0, 1): cov_rows[1], (0, 2): cov_rows[2],
         (1, 1): cov_rows[4], (1, 2): cov_rows[5], (2, 2): cov_rows[8]}
    one = jnp.ones_like(cov_rows[0])
    zero = jnp.zeros_like(cov_rows[0])
    v = {(r, c): (one if r == c else zero) for r in range(3) for c in range(3)}

    def A(r, c):
        return a[(r, c)] if r <= c else a[(c, r)]

    for _ in range(6):
        for (p, q) in ((0, 1), (0, 2), (1, 2)):
            apq = A(p, q)
            app = A(p, p)
            aqq = A(q, q)
            tau = (aqq - app) / (2.0 * apq)
            t = jnp.sign(tau) / (jnp.abs(tau) + jnp.sqrt(1.0 + tau * tau))
            t = jnp.where(apq == 0.0, 0.0, t)
            c_ = 1.0 / jnp.sqrt(1.0 + t * t)
            s_ = t * c_
            r = 3 - p - q
            apr, aqr = A(p, r), A(q, r)
            a[(p, p)] = app - t * apq
            a[(q, q)] = aqq + t * apq
            a[(p, q)] = zero
            a[(min(p, r), max(p, r))] = c_ * apr - s_ * aqr
            a[(min(q, r), max(q, r))] = s_ * apr + c_ * aqr
            for i3 in range(3):
                vip, viq = v[(i3, p)], v[(i3, q)]
                v[(i3, p)] = c_ * vip - s_ * viq
                v[(i3, q)] = s_ * vip + c_ * viq

    l0, l1, l2 = a[(0, 0)], a[(1, 1)], a[(2, 2)]
    is0 = (l0 <= l1) & (l0 <= l2)
    is1 = jnp.logical_not(is0) & (l1 <= l2)

    def pick(r):
        return jnp.where(is0, v[(r, 0)], jnp.where(is1, v[(r, 1)], v[(r, 2)]))

    rn = [pick(0), pick(1), pick(2)]
    dotn = rn[0] * nT[0:1, :] + rn[1] * nT[1:2, :] + rn[2] * nT[2:3, :]
    sgn = jnp.where(dotn < 0.0, -1.0, 1.0)
    return [rn[0] * sgn, rn[1] * sgn, rn[2] * sgn]


PLN_R = 2000


def _planes_body(pts_ref, nT_ref, dist_ref, cnt_ref, s1_ref,
                 out_ref, cov_s, rn_s, rd_s, val_s):
    i = pl.program_id(0)
    nblk = NPTS // PLN_R
    pts = pts_ref[...]
    nT = nT_ref[...]
    pd = jnp.zeros((PLN_R, NPLANES), jnp.float32) - dist_ref[...]
    for c in range(3):
        pd = pd + pts[:, c:c + 1] * nT[c:c + 1, :]
    m = (jnp.abs(pd) < PLANE_THR).astype(jnp.float32)

    @pl.when(i == 0)
    def _():
        cov_s[...] = jnp.zeros_like(cov_s)

    @pl.when(i < nblk)
    def _():
        cnt = jnp.maximum(cnt_ref[...], 1.0)
        cen = [(pts[:, c:c + 1] - s1_ref[c:c + 1, :] / cnt) * m
               for c in range(3)]
        j = 0
        for aa in range(3):
            for bb in range(3):
                cov_s[j, :] += jnp.sum(cen[aa] * cen[bb], axis=0)
                j += 1

    @pl.when(i == nblk)
    def _():
        cnt = jnp.maximum(cnt_ref[...], 1.0)
        ct = [s1_ref[c:c + 1, :] / cnt for c in range(3)]
        rn = _jacobi_smallest([cov_s[j:j + 1, :] for j in range(9)],
                              nT_ref[...])
        rd = ct[0] * rn[0] + ct[1] * rn[1] + ct[2] * rn[2]
        for c in range(3):
            rn_s[c:c + 1, :] = rn[c]
        rd_s[...] = rd
        val_s[...] = (cnt_ref[...] >= 3.0).astype(jnp.float32)

    @pl.when(i >= nblk)
    def _():
        rnT = rn_s[...]
        rd = rd_s[...]
        valid = val_s[...]
        px = pts[:, 0]
        py = pts[:, 1]
        pz = pts[:, 2]
        for p in range(NPLANES):
            coef = valid[0, p] * m[:, p]
            dot = px * rnT[0, p] + py * rnT[1, p] + pz * rnT[2, p]
            scale = coef * (dot - rd[0, p])
            px = px - scale * rnT[0, p]
            py = py - scale * rnT[1, p]
            pz = pz - scale * rnT[2, p]
        out_ref[...] = jnp.concatenate(
            [px[:, None], py[:, None], pz[:, None]], axis=1)


def _planes_call(pts, nT, dist, cnt2d, s1):
    nblk = NPTS // PLN_R
    return pl.pallas_call(
        _planes_body,
        grid=(2 * nblk,),
        in_specs=[
            pl.BlockSpec((PLN_R, 3), lambda i: (lax.rem(i, nblk), 0)),
            pl.BlockSpec((3, NPLANES), lambda i: (0, 0)),
            pl.BlockSpec((1, NPLANES), lambda i: (0, 0)),
            pl.BlockSpec((1, NPLANES), lambda i: (0, 0)),
            pl.BlockSpec((3, NPLANES), lambda i: (0, 0)),
        ],
        out_specs=pl.BlockSpec((PLN_R, 3),
                               lambda i: (jnp.maximum(i - nblk, 0), 0)),
        out_shape=jax.ShapeDtypeStruct((NPTS, 3), jnp.float32),
        scratch_shapes=[
            pltpu.VMEM((9, NPLANES), jnp.float32),
            pltpu.VMEM((3, NPLANES), jnp.float32),
            pltpu.VMEM((1, NPLANES), jnp.float32),
            pltpu.VMEM((1, NPLANES), jnp.float32),
        ],
    )(pts, nT, dist, cnt2d, s1)


def kernel(points, normals, distances, w1a, b1a, w1b, b1b, w2a, b2a, w2b, b2b,
           w3a, b3a, w3b, b3b, w4, b4, w5, b5):
    f32 = jnp.float32
    pointsT = points.T
    wA1, wB1 = w1a[:3] - w1a[3:], w1a[3:]
    wA2, wB2 = w2a[:64] - w2a[64:], w2a[64:]
    wA3, wB3 = w3a[:64] - w3a[64:], w3a[64:]

    nbrs, ax1, bx1 = _knn_call(points, pointsT, wA1, wB1, b1a[None, :])
    idx2d = nbrs.reshape(IDX_ROWS, 128)

    g1 = _sc_gather(bx1, idx2d).reshape(NPTS, KNN, 64)
    f1, ax2, bx2 = _convB_A_call(ax1, g1, w1b, b1b[None, :],
                                 wA2, wB2, b2a[None, :])
    g2 = _sc_gather(bx2, idx2d).reshape(NPTS, KNN, 64)
    f2, ax3, bx3 = _convB_A_call(ax2, g2, w2b, b2b[None, :],
                                 wA3, wB3, b3a[None, :])
    g3 = _sc_gather(bx3, idx2d).reshape(NPTS, KNN, 64)

    nT = normals.T.astype(f32)
    dist = distances[None, :].astype(f32)
    pts, cnt2d, s1 = _final_call(
        ax3, g3, w3b, b3b[None, :], f1, f2,
        w4[0:64], w4[64:128], w4[128:192], b4[None, :], w5, b5[None, :],
        points, nT, dist)

    return _planes_call(pts, nT, dist, cnt2d, s1)


# speedup vs baseline: 6.8434x; 1.0094x over previous
import functools

import jax
import jax.numpy as jnp
from jax import lax
from jax.experimental import pallas as pl
from jax.experimental.pallas import tpu as pltpu
from jax.experimental.pallas import tpu_sc as plsc

NPTS = 10000
KNN = 16
NPLANES = 8
PLANE_THR = 0.05


KNN_R = 80


def _knn_body(pts_ref, ptsT_ref, wA_ref, wB_ref, ba_ref,
              nbr_ref, ax_ref, bx_ref):
    i = pl.program_id(0)
    pts_r = pts_ref[...]
    ptsT = ptsT_ref[...]

    sq_r = jnp.zeros((KNN_R, 1), jnp.float32)
    sq_all = jnp.zeros((1, NPTS), jnp.float32)
    for c in range(3):
        sq_r = sq_r + pts_r[:, c:c + 1] * pts_r[:, c:c + 1]
        sq_all = sq_all + ptsT[c:c + 1, :] * ptsT[c:c + 1, :]
    dot = jnp.dot(pts_r.astype(jnp.bfloat16), ptsT.astype(jnp.bfloat16),
                  preferred_element_type=jnp.float32)
    d = (sq_r + sq_all) - 2.0 * dot

    colf = lax.broadcasted_iota(jnp.int32, (KNN_R, NPTS), 1).astype(jnp.float32)
    row_global = (lax.broadcasted_iota(jnp.int32, (KNN_R, NPTS), 0)
                  .astype(jnp.float32) + jnp.float32(i * KNN_R))
    inf = jnp.float32(jnp.inf)
    d = jnp.where(colf == row_global, inf, d)

    bigf = jnp.float32(2**30)
    cols_out = []
    minv = jnp.min(d, axis=1, keepdims=True)
    for k in range(KNN):
        eq = d == minv
        idxf = jnp.min(jnp.where(eq, colf, bigf), axis=1)
        cols_out.append(idxf[:, None])
        if k < KNN - 1:
            d = jnp.where(eq, inf, d)
            minv = jnp.min(d, axis=1, keepdims=True)
    nbr_ref[...] = jnp.concatenate(cols_out, axis=1).astype(jnp.int32)

    ax = jnp.zeros((KNN_R, 64), jnp.float32) + ba_ref[...]
    bx = jnp.zeros((KNN_R, 64), jnp.float32)
    for c in range(3):
        ax = ax + pts_r[:, c:c + 1] * wA_ref[c:c + 1, :]
        bx = bx + pts_r[:, c:c + 1] * wB_ref[c:c + 1, :]
    ax_ref[...] = ax
    bx_ref[...] = bx


def _knn_call(points, pointsT, wA1, wB1, b1a):
    nblk = NPTS // KNN_R
    return pl.pallas_call(
        _knn_body,
        grid=(nblk,),
        in_specs=[
            pl.BlockSpec((KNN_R, 3), lambda i: (i, 0)),
            pl.BlockSpec((3, NPTS), lambda i: (0, 0)),
            pl.BlockSpec((3, 64), lambda i: (0, 0)),
            pl.BlockSpec((3, 64), lambda i: (0, 0)),
            pl.BlockSpec((1, 64), lambda i: (0, 0)),
        ],
        out_specs=[
            pl.BlockSpec((KNN_R, KNN), lambda i: (i, 0)),
            pl.BlockSpec((KNN_R, 64), lambda i: (i, 0)),
            pl.BlockSpec((KNN_R, 64), lambda i: (i, 0)),
        ],
        out_shape=[
            jax.ShapeDtypeStruct((NPTS, KNN), jnp.int32),
            jax.ShapeDtypeStruct((NPTS, 64), jnp.float32),
            jax.ShapeDtypeStruct((NPTS, 64), jnp.float32),
        ],
    )(points, pointsT, wA1, wB1, b1a)


IDX_ROWS = (NPTS * KNN) // 128


def _sc_gather(table, idx2d):
    info = plsc.get_sparse_core_info()
    nc, ns = info.num_cores, info.num_subcores
    nw = nc * ns
    jmax = (IDX_ROWS + nw - 1) // nw
    mesh = plsc.VectorSubcoreMesh(core_axis_name="c", subcore_axis_name="s")

    @functools.partial(
        pl.kernel, mesh=mesh,
        compiler_params=pltpu.CompilerParams(use_tc_tiling_on_sc=False),
        out_type=jax.ShapeDtypeStruct((IDX_ROWS, 128, 64), jnp.float32),
        scratch_types=[
            pltpu.VMEM((2, 128), jnp.int32),
            pltpu.VMEM((2, 128, 64), jnp.float32),
            pltpu.SemaphoreType.DMA,
            pltpu.SemaphoreType.DMA,
            pltpu.SemaphoreType.DMA,
            pltpu.SemaphoreType.DMA,
        ],
    )
    def gk(table_hbm, idx_hbm, out_hbm, idx_v, rows_v, g0, g1, o0, o1):
        w = lax.axis_index("s") * nc + lax.axis_index("c")
        gsem = (g0, g1)
        osem = (o0, o1)

        def start_gather(j, b):
            pltpu.sync_copy(idx_hbm.at[j * nw + w], idx_v.at[b])
            pltpu.async_copy(table_hbm.at[idx_v.at[b]], rows_v.at[b], gsem[b])

        def wait_gather(b):
            pltpu.make_async_copy(table_hbm.at[idx_v.at[b]], rows_v.at[b],
                                  gsem[b]).wait()

        def start_out(j, b):
            pltpu.async_copy(rows_v.at[b], out_hbm.at[j * nw + w], osem[b])

        def wait_out(j, b):
            pltpu.make_async_copy(rows_v.at[b], out_hbm.at[j * nw + w],
                                  osem[b]).wait()

        start_gather(0, 0)

        def body(g, carry):
            for b in range(2):
                j = g * 2 + b
                jn = j + 1
                bn = 1 - b

                @pl.when(jn * nw + w < IDX_ROWS)
                def _():
                    @pl.when(j >= 1)
                    def _():
                        wait_out(j - 1, bn)
                    start_gather(jn, bn)

                @pl.when(j * nw + w < IDX_ROWS)
                def _():
                    wait_gather(b)
                    start_out(j, b)
            return carry

        lax.fori_loop(0, jmax // 2, body, 0)
        wait_out(jmax - 2, 0)

        @pl.when((jmax - 1) * nw + w < IDX_ROWS)
        def _():
            wait_out(jmax - 1, 1)

    return gk(table, idx2d)


CONV_R = 1000


def _convB_A_body(ax_ref, g_ref, wb_ref, bb_ref, wAn_ref, wBn_ref, ban_ref,
                  f_ref, axn_ref, bxn_ref):
    ax = ax_ref[...]
    g = g_ref[...]
    h1 = jax.nn.relu(ax[:, None, :] + g)
    h1f = h1.reshape(CONV_R * KNN, 64)
    h2 = jnp.dot(h1f, wb_ref[...],
                 preferred_element_type=jnp.float32) + bb_ref[...]
    f = jnp.max(h2.reshape(CONV_R, KNN, 64), axis=1)
    f_ref[...] = f
    axn_ref[...] = jnp.dot(f, wAn_ref[...],
                           preferred_element_type=jnp.float32) + ban_ref[...]
    bxn_ref[...] = jnp.dot(f, wBn_ref[...], preferred_element_type=jnp.float32)


def _convB_A_call(ax, g3, wb, bb, wAn, wBn, ban):
    nblk = NPTS // CONV_R
    return pl.pallas_call(
        _convB_A_body,
        grid=(nblk,),
        in_specs=[
            pl.BlockSpec((CONV_R, 64), lambda i: (i, 0)),
            pl.BlockSpec((CONV_R, KNN, 64), lambda i: (i, 0, 0)),
            pl.BlockSpec((64, 64), lambda i: (0, 0)),
            pl.BlockSpec((1, 64), lambda i: (0, 0)),
            pl.BlockSpec((64, 64), lambda i: (0, 0)),
            pl.BlockSpec((64, 64), lambda i: (0, 0)),
            pl.BlockSpec((1, 64), lambda i: (0, 0)),
        ],
        out_specs=[
            pl.BlockSpec((CONV_R, 64), lambda i: (i, 0)),
            pl.BlockSpec((CONV_R, 64), lambda i: (i, 0)),
            pl.BlockSpec((CONV_R, 64), lambda i: (i, 0)),
        ],
        out_shape=[
            jax.ShapeDtypeStruct((NPTS, 64), jnp.float32),
            jax.ShapeDtypeStruct((NPTS, 64), jnp.float32),
            jax.ShapeDtypeStruct((NPTS, 64), jnp.float32),
        ],
    )(ax, g3, wb, bb, wAn, wBn, ban)


def _final_body(ax_ref, g_ref, wb_ref, bb_ref, f1_ref, f2_ref,
                w4a_ref, w4b_ref, w4c_ref, b4_ref, w5_ref, b5_ref,
                pts_ref, nT_ref, dist_ref,
                out_ref, cnt_ref, s1_ref):
    i = pl.program_id(0)
    ax = ax_ref[...]
    g = g_ref[...]
    h1 = jax.nn.relu(ax[:, None, :] + g)
    h1f = h1.reshape(CONV_R * KNN, 64)
    h2 = jnp.dot(h1f, wb_ref[...],
                 preferred_element_type=jnp.float32) + bb_ref[...]
    f3 = jnp.max(h2.reshape(CONV_R, KNN, 64), axis=1)

    t = (jnp.dot(f1_ref[...], w4a_ref[...], preferred_element_type=jnp.float32)
         + jnp.dot(f2_ref[...], w4b_ref[...], preferred_element_type=jnp.float32)
         + jnp.dot(f3, w4c_ref[...], preferred_element_type=jnp.float32)
         + b4_ref[...])
    t = jax.nn.relu(t)
    res = jnp.dot(t, w5_ref[...], preferred_element_type=jnp.float32) + b5_ref[...]
    pts = pts_ref[...] + res
    out_ref[...] = pts

    nT = nT_ref[...]
    pd = jnp.zeros((CONV_R, NPLANES), jnp.float32) - dist_ref[...]
    for c in range(3):
        pd = pd + pts[:, c:c + 1] * nT[c:c + 1, :]
    m = (jnp.abs(pd) < PLANE_THR).astype(jnp.float32)

    @pl.when(i == 0)
    def _():
        cnt_ref[...] = jnp.zeros_like(cnt_ref)
        s1_ref[...] = jnp.zeros_like(s1_ref)

    cnt_ref[0, :] += jnp.sum(m, axis=0)
    for c in range(3):
        s1_ref[c, :] += jnp.sum(m * pts[:, c:c + 1], axis=0)


def _final_call(ax3, g3, w3b, b3b, f1, f2, w4a, w4b, w4c, b4, w5, b5,
                points, nT, dist):
    nblk = NPTS // CONV_R
    return pl.pallas_call(
        _final_body,
        grid=(nblk,),
        in_specs=[
            pl.BlockSpec((CONV_R, 64), lambda i: (i, 0)),
            pl.BlockSpec((CONV_R, KNN, 64), lambda i: (i, 0, 0)),
            pl.BlockSpec((64, 64), lambda i: (0, 0)),
            pl.BlockSpec((1, 64), lambda i: (0, 0)),
            pl.BlockSpec((CONV_R, 64), lambda i: (i, 0)),
            pl.BlockSpec((CONV_R, 64), lambda i: (i, 0)),
            pl.BlockSpec((64, 256), lambda i: (0, 0)),
            pl.BlockSpec((64, 256), lambda i: (0, 0)),
            pl.BlockSpec((64, 256), lambda i: (0, 0)),
            pl.BlockSpec((1, 256), lambda i: (0, 0)),
            pl.BlockSpec((256, 3), lambda i: (0, 0)),
            pl.BlockSpec((1, 3), lambda i: (0, 0)),
            pl.BlockSpec((CONV_R, 3), lambda i: (i, 0)),
            pl.BlockSpec((3, NPLANES), lambda i: (0, 0)),
            pl.BlockSpec((1, NPLANES), lambda i: (0, 0)),
        ],
        out_specs=[
            pl.BlockSpec((CONV_R, 3), lambda i: (i, 0)),
            pl.BlockSpec((1, NPLANES), lambda i: (0, 0)),
            pl.BlockSpec((3, NPLANES), lambda i: (0, 0)),
        ],
        out_shape=[
            jax.ShapeDtypeStruct((NPTS, 3), jnp.float32),
            jax.ShapeDtypeStruct((1, NPLANES), jnp.float32),
            jax.ShapeDtypeStruct((3, NPLANES), jnp.float32),
        ],
    )(ax3, g3, w3b, b3b, f1, f2, w4a, w4b, w4c, b4, w5, b5, points, nT, dist)


def _jacobi_smallest(cov_rows, nT):
    a = {(0, 0): cov_rows[0], (0, 1): cov_rows[1], (0, 2): cov_rows[2],
         (1, 1): cov_rows[4], (1, 2): cov_rows[5], (2, 2): cov_rows[8]}
    one = jnp.ones_like(cov_rows[0])
    zero = jnp.zeros_like(cov_rows[0])
    v = {(r, c): (one if r == c else zero) for r in range(3) for c in range(3)}

    def A(r, c):
        return a[(r, c)] if r <= c else a[(c, r)]

    for _ in range(6):
        for (p, q) in ((0, 1), (0, 2), (1, 2)):
            apq = A(p, q)
            app = A(p, p)
            aqq = A(q, q)
            tau = (aqq - app) / (2.0 * apq)
            t = jnp.sign(tau) / (jnp.abs(tau) + jnp.sqrt(1.0 + tau * tau))
            t = jnp.where(apq == 0.0, 0.0, t)
            c_ = 1.0 / jnp.sqrt(1.0 + t * t)
            s_ = t * c_
            r = 3 - p - q
            apr, aqr = A(p, r), A(q, r)
            a[(p, p)] = app - t * apq
            a[(q, q)] = aqq + t * apq
            a[(p, q)] = zero
            a[(min(p, r), max(p, r))] = c_ * apr - s_ * aqr
            a[(min(q, r), max(q, r))] = s_ * apr + c_ * aqr
            for i3 in range(3):
                vip, viq = v[(i3, p)], v[(i3, q)]
                v[(i3, p)] = c_ * vip - s_ * viq
                v[(i3, q)] = s_ * vip + c_ * viq

    l0, l1, l2 = a[(0, 0)], a[(1, 1)], a[(2, 2)]
    is0 = (l0 <= l1) & (l0 <= l2)
    is1 = jnp.logical_not(is0) & (l1 <= l2)

    def pick(r):
        return jnp.where(is0, v[(r, 0)], jnp.where(is1, v[(r, 1)], v[(r, 2)]))

    rn = [pick(0), pick(1), pick(2)]
    dotn = rn[0] * nT[0:1, :] + rn[1] * nT[1:2, :] + rn[2] * nT[2:3, :]
    sgn = jnp.where(dotn < 0.0, -1.0, 1.0)
    return [rn[0] * sgn, rn[1] * sgn, rn[2] * sgn]


PLN_R = 2000


def _planes_body(pts_ref, nT_ref, dist_ref, cnt_ref, s1_ref,
                 out_ref, cov_s, rn_s, rd_s, val_s):
    i = pl.program_id(0)
    nblk = NPTS // PLN_R
    pts = pts_ref[...]
    nT = nT_ref[...]
    pd = jnp.zeros((PLN_R, NPLANES), jnp.float32) - dist_ref[...]
    for c in range(3):
        pd = pd + pts[:, c:c + 1] * nT[c:c + 1, :]
    m = (jnp.abs(pd) < PLANE_THR).astype(jnp.float32)

    @pl.when(i == 0)
    def _():
        cov_s[...] = jnp.zeros_like(cov_s)

    @pl.when(i < nblk)
    def _():
        cnt = jnp.maximum(cnt_ref[...], 1.0)
        cen = [(pts[:, c:c + 1] - s1_ref[c:c + 1, :] / cnt) * m
               for c in range(3)]
        j = 0
        for aa in range(3):
            for bb in range(3):
                cov_s[j, :] += jnp.sum(cen[aa] * cen[bb], axis=0)
                j += 1

    @pl.when(i == nblk)
    def _():
        cnt = jnp.maximum(cnt_ref[...], 1.0)
        ct = [s1_ref[c:c + 1, :] / cnt for c in range(3)]
        rn = _jacobi_smallest([cov_s[j:j + 1, :] for j in range(9)],
                              nT_ref[...])
        rd = ct[0] * rn[0] + ct[1] * rn[1] + ct[2] * rn[2]
        for c in range(3):
            rn_s[c:c + 1, :] = rn[c]
        rd_s[...] = rd
        val_s[...] = (cnt_ref[...] >= 3.0).astype(jnp.float32)

    @pl.when(i >= nblk)
    def _():
        rnT = rn_s[...]
        rd = rd_s[...]
        valid = val_s[...]
        px = pts[:, 0]
        py = pts[:, 1]
        pz = pts[:, 2]
        for p in range(NPLANES):
            coef = valid[0, p] * m[:, p]
            dot = px * rnT[0, p] + py * rnT[1, p] + pz * rnT[2, p]
            scale = coef * (dot - rd[0, p])
            px = px - scale * rnT[0, p]
            py = py - scale * rnT[1, p]
            pz = pz - scale * rnT[2, p]
        out_ref[...] = jnp.concatenate(
            [px[:, None], py[:, None], pz[:, None]], axis=1)


def _planes_call(pts, nT, dist, cnt2d, s1):
    nblk = NPTS // PLN_R
    return pl.pallas_call(
        _planes_body,
        grid=(2 * nblk,),
        in_specs=[
            pl.BlockSpec((PLN_R, 3), lambda i: (lax.rem(i, nblk), 0)),
            pl.BlockSpec((3, NPLANES), lambda i: (0, 0)),
            pl.BlockSpec((1, NPLANES), lambda i: (0, 0)),
            pl.BlockSpec((1, NPLANES), lambda i: (0, 0)),
            pl.BlockSpec((3, NPLANES), lambda i: (0, 0)),
        ],
        out_specs=pl.BlockSpec((PLN_R, 3),
                               lambda i: (jnp.maximum(i - nblk, 0), 0)),
        out_shape=jax.ShapeDtypeStruct((NPTS, 3), jnp.float32),
        scratch_shapes=[
            pltpu.VMEM((9, NPLANES), jnp.float32),
            pltpu.VMEM((3, NPLANES), jnp.float32),
            pltpu.VMEM((1, NPLANES), jnp.float32),
            pltpu.VMEM((1, NPLANES), jnp.float32),
        ],
    )(pts, nT, dist, cnt2d, s1)


def kernel(points, normals, distances, w1a, b1a, w1b, b1b, w2a, b2a, w2b, b2b,
           w3a, b3a, w3b, b3b, w4, b4, w5, b5):
    f32 = jnp.float32
    pointsT = points.T
    wA1, wB1 = w1a[:3] - w1a[3:], w1a[3:]
    wA2, wB2 = w2a[:64] - w2a[64:], w2a[64:]
    wA3, wB3 = w3a[:64] - w3a[64:], w3a[64:]

    nbrs, ax1, bx1 = _knn_call(points, pointsT, wA1, wB1, b1a[None, :])
    idx2d = nbrs.reshape(IDX_ROWS, 128)

    g1 = _sc_gather(bx1, idx2d).reshape(NPTS, KNN, 64)
    f1, ax2, bx2 = _convB_A_call(ax1, g1, w1b, b1b[None, :],
                                 wA2, wB2, b2a[None, :])
    g2 = _sc_gather(bx2, idx2d).reshape(NPTS, KNN, 64)
    f2, ax3, bx3 = _convB_A_call(ax2, g2, w2b, b2b[None, :],
                                 wA3, wB3, b3a[None, :])
    g3 = _sc_gather(bx3, idx2d).reshape(NPTS, KNN, 64)

    nT = normals.T.astype(f32)
    dist = distances[None, :].astype(f32)
    pts, cnt2d, s1 = _final_call(
        ax3, g3, w3b, b3b[None, :], f1, f2,
        w4[0:64], w4[64:128], w4[128:192], b4[None, :], w5, b5[None, :],
        points, nT, dist)

    return _planes_call(pts, nT, dist, cnt2d, s1)
